# baseline (speedup 1.0000x reference)
import functools

import jax
import jax.numpy as jnp
from jax import lax
from jax.experimental import pallas as pl
from jax.experimental.pallas import tpu as pltpu
from jax.experimental.pallas import tpu_sc as plsc

N = 10000
E = 320000
NC = 2
NS = 16
NW = NC * NS
EW = E // NW
ROWS_PER_TILE = N // NS

F32 = jnp.float32
I32 = jnp.int32


_TAKE_DNUMS = lax.GatherDimensionNumbers(
    offset_dims=(), collapsed_slice_dims=(0,), start_index_map=(0,))


def _take16(v, lane):
    idx = jnp.full((16, 1), lane, dtype=I32)
    return lax.gather(v, idx, _TAKE_DNUMS, slice_sizes=(1,),
                      mode=lax.GatherScatterMode.PROMISE_IN_BOUNDS)


def _iota16():
    return lax.iota(I32, 16)


_C1 = 80
_NCH1 = EW // _C1
_I1 = 41


def _sc_conv1_body(srcr, dstr, tsrc, tdst, out,
                   sidxb, didxb, sb0, sb1, sb2, db0, db1, db2, accum,
                   gs0, gs1, gs2, ds0, ds1, ds2, ss0, ss1, ss2,
                   is0, is1, is2, is3, is4, is5, zsem):
    c = lax.axis_index("c")
    s = lax.axis_index("s")
    wid = s * NC + c
    sbufs = (sb0, sb1, sb2)
    dbufs = (db0, db1, db2)
    gsems = (gs0, gs1, gs2)
    dsems = (ds0, ds1, ds2)
    ssems = (ss0, ss1, ss2)
    isems = (is0, is1, is2, is3, is4, is5)

    def i_start(ci, j):
        pltpu.async_copy(srcr.at[wid, ci], sidxb.at[j], isems[j])
        pltpu.async_copy(dstr.at[wid, ci], didxb.at[j], isems[j])

    def i_wait(ci, j):
        pltpu.make_async_copy(srcr.at[wid, ci], sidxb.at[j], isems[j]).wait()
        pltpu.make_async_copy(dstr.at[wid, ci], didxb.at[j], isems[j]).wait()

    def g_start(ci, k, j):
        pltpu.async_copy(tsrc.at[sidxb.at[j]], sbufs[k], gsems[k])
        pltpu.async_copy(tdst.at[didxb.at[j]], dbufs[k], dsems[k])

    def g_wait(ci, k, j):
        pltpu.make_async_copy(tsrc.at[sidxb.at[j]], sbufs[k], gsems[k]).wait()
        pltpu.make_async_copy(tdst.at[didxb.at[j]], dbufs[k], dsems[k]).wait()

    def s_start(ci, k, j):
        pltpu.async_copy(sbufs[k], accum.at[didxb.at[j]], ssems[k], add=False)

    def s_wait(ci, k, j):
        pltpu.make_async_copy(sbufs[k], accum.at[didxb.at[j]], ssems[k]).wait()

    def compute(k):
        sb = sbufs[k]
        db = dbufs[k]
        for e in range(_C1):
            a_s = sb[e, pl.ds(128, 16)]
            ad = db[e, pl.ds(0, 16)]
            t = a_s + ad
            w = jnp.exp(jnp.maximum(t, 0.2 * t))
            sb[e, pl.ds(128, 16)] = w
            for h in range(8):
                sp = _take16(w, h)
                sb[e, pl.ds(h * 16, 16)] = sb[e, pl.ds(h * 16, 16)] * sp

    i_start(0, 0)
    i_start(1, 1)
    i_start(2, 2)
    z16 = jnp.zeros((16,), F32)
    for e in range(_C1):
        for l in range(9):
            sb2[e, pl.ds(l * 16, 16)] = z16

    @pl.when(s < 15)
    def _():
        for j in range(8):
            pltpu.async_copy(sb2, accum.at[pl.ds(s * 640 + j * 80, 80)], zsem)
        for j in range(8):
            pltpu.make_async_copy(sb2, accum.at[pl.ds(s * 640 + j * 80, 80)], zsem).wait()

    @pl.when(s == 15)
    def _():
        for j in range(5):
            pltpu.async_copy(sb2, accum.at[pl.ds(9600 + j * 80, 80)], zsem)
        for j in range(5):
            pltpu.make_async_copy(sb2, accum.at[pl.ds(9600 + j * 80, 80)], zsem).wait()

    i_wait(0, 0)
    g_start(0, 0, 0)
    plsc.subcore_barrier()

    def do_third(a, i, S, P):
        i_start(a + 3, P[0])

        @pl.when(i > 0)
        def _():
            s_wait(a - 2, 1, P[1])

        i_start(a + 4, P[1])
        i_wait(a + 1, S[1])
        g_start(a + 1, 1, S[1])
        g_wait(a, 0, S[0])
        compute(0)
        s_start(a, 0, S[0])

        @pl.when(i > 0)
        def _():
            s_wait(a - 1, 2, P[2])

        @pl.when(i < _I1 - 1)
        def _():
            i_start(a + 5, P[2])

        i_wait(a + 2, S[2])
        g_start(a + 2, 2, S[2])
        g_wait(a + 1, 1, S[1])
        compute(1)
        s_start(a + 1, 1, S[1])
        s_wait(a, 0, S[0])
        i_wait(a + 3, P[0])
        g_start(a + 3, 0, P[0])
        g_wait(a + 2, 2, S[2])
        compute(2)
        s_start(a + 2, 2, S[2])

    def body(i, carry):
        a = 3 * i
        par = lax.rem(i, 2)

        @pl.when(par == 0)
        def _():
            do_third(a, i, (0, 1, 2), (3, 4, 5))

        @pl.when(par == 1)
        def _():
            do_third(a, i, (3, 4, 5), (0, 1, 2))

        return carry

    lax.fori_loop(0, _I1, body, 0)

    s_wait(121, 1, 1)
    i_wait(124, 4)
    g_start(124, 1, 4)
    g_wait(123, 0, 3)
    compute(0)
    s_start(123, 0, 3)
    g_wait(124, 1, 4)
    compute(1)
    s_start(124, 1, 4)
    s_wait(122, 2, 2)
    s_wait(123, 0, 3)
    s_wait(124, 1, 4)

    plsc.subcore_barrier()

    @pl.when(s < 15)
    def _():
        pltpu.sync_copy(accum.at[pl.ds(s * 640, 640)],
                        out.at[c, pl.ds(s * 640, 640)])

    @pl.when(s == 15)
    def _():
        pltpu.sync_copy(accum.at[pl.ds(9600, 400)],
                        out.at[c, pl.ds(9600, 400)])


def _sc_conv1(srcr, dstr, tsrc, tdst):
    mesh = plsc.VectorSubcoreMesh(core_axis_name="c", subcore_axis_name="s",
                                  num_cores=NC, num_subcores=NS)
    f = functools.partial(
        pl.kernel,
        out_type=jax.ShapeDtypeStruct((NC, N, 144), F32),
        mesh=mesh,
        compiler_params=pltpu.CompilerParams(use_tc_tiling_on_sc=False, needs_layout_passes=False),
        scratch_types=[
            pltpu.VMEM((6, _C1), I32),
            pltpu.VMEM((6, _C1), I32),
            pltpu.VMEM((_C1, 144), F32),
            pltpu.VMEM((_C1, 144), F32),
            pltpu.VMEM((_C1, 144), F32),
            pltpu.VMEM((_C1, 16), F32),
            pltpu.VMEM((_C1, 16), F32),
            pltpu.VMEM((_C1, 16), F32),
            pltpu.VMEM_SHARED((N, 144), F32),
            pltpu.SemaphoreType.DMA,
            pltpu.SemaphoreType.DMA,
            pltpu.SemaphoreType.DMA,
            pltpu.SemaphoreType.DMA,
            pltpu.SemaphoreType.DMA,
            pltpu.SemaphoreType.DMA,
            pltpu.SemaphoreType.DMA,
            pltpu.SemaphoreType.DMA,
            pltpu.SemaphoreType.DMA,
            pltpu.SemaphoreType.DMA,
            pltpu.SemaphoreType.DMA,
            pltpu.SemaphoreType.DMA,
            pltpu.SemaphoreType.DMA,
            pltpu.SemaphoreType.DMA,
            pltpu.SemaphoreType.DMA,
            pltpu.SemaphoreType.DMA,
        ],
    )(_sc_conv1_body)
    return f(srcr, dstr, tsrc, tdst)


_C2 = 400
_NCH2 = EW // _C2
_I2 = 8


def _sc_conv2_body(srcr, dstr, txw, as2h, ad2h, out,
                   sidx, didx, sb0, sb1, sb2, mb0, mb1, mb2, asv, adv, zbuf, accum,
                   gs0, gs1, gs2, ss0, ss1, ss2, zsem):
    c = lax.axis_index("c")
    s = lax.axis_index("s")
    wid = s * NC + c
    sbufs = (sb0, sb1, sb2)
    mbufs = (mb0, mb1, mb2)
    gsems = (gs0, gs1, gs2)
    ssems = (ss0, ss1, ss2)

    pltpu.sync_copy(srcr.at[wid], sidx)
    pltpu.sync_copy(dstr.at[wid], didx)
    pltpu.sync_copy(as2h, asv)
    pltpu.sync_copy(ad2h, adv)

    z16 = jnp.zeros((16,), F32)
    for r in range(40):
        for l in range(2):
            zbuf[r, pl.ds(l * 16, 16)] = z16

    @pl.when(s < 15)
    def _():
        for j in range(16):
            pltpu.async_copy(zbuf, accum.at[pl.ds(s * 640 + j * 40, 40)], zsem)
        for j in range(16):
            pltpu.make_async_copy(zbuf, accum.at[pl.ds(s * 640 + j * 40, 40)], zsem).wait()

    @pl.when(s == 15)
    def _():
        for j in range(10):
            pltpu.async_copy(zbuf, accum.at[pl.ds(9600 + j * 40, 40)], zsem)
        for j in range(10):
            pltpu.make_async_copy(zbuf, accum.at[pl.ds(9600 + j * 40, 40)], zsem).wait()

    def g_start(ci, k):
        pltpu.async_copy(txw.at[sidx.at[ci]], sbufs[k], gsems[k])

    def g_wait(ci, k):
        pltpu.make_async_copy(txw.at[sidx.at[ci]], sbufs[k], gsems[k]).wait()

    def s_start(ci, k):
        pltpu.async_copy(mbufs[k], accum.at[didx.at[ci]], ssems[k], add=True)

    def s_wait(ci, k):
        pltpu.make_async_copy(mbufs[k], accum.at[didx.at[ci]], ssems[k]).wait()

    iota = _iota16()
    oh0 = jnp.where(iota == 0, 1.0, 0.0).astype(F32)
    oh1 = jnp.where(iota == 1, 1.0, 0.0).astype(F32)

    def compute(ci, k):
        sb = sbufs[k]
        mb = mbufs[k]
        ci16 = jnp.full((16,), ci, dtype=I32)
        for grp in range(_C2 // 16):
            col = iota + grp * 16
            s16 = plsc.load_gather(sidx, [ci16, col])
            d16 = plsc.load_gather(didx, [ci16, col])
            a_s = plsc.load_gather(asv, [s16])
            a_d = plsc.load_gather(adv, [d16])
            t = a_s + a_d
            w = jnp.exp(jnp.maximum(t, 0.2 * t))
            for e in range(16):
                r = grp * 16 + e
                sp = _take16(w, e)
                mb[r, pl.ds(0, 16)] = sb[r, pl.ds(0, 16)] * sp
                mb[r, pl.ds(16, 16)] = sp * oh0 + oh1

    g_start(0, 0)
    plsc.subcore_barrier()

    def body(i, carry):
        a = 3 * i

        @pl.when(i > 0)
        def _():
            s_wait(a - 2, 1)

        g_start(a + 1, 1)
        g_wait(a, 0)
        compute(a, 0)
        s_start(a, 0)

        @pl.when(i > 0)
        def _():
            s_wait(a - 1, 2)

        g_start(a + 2, 2)
        g_wait(a + 1, 1)
        compute(a + 1, 1)
        s_start(a + 1, 1)

        s_wait(a, 0)
        g_start(a + 3, 0)
        g_wait(a + 2, 2)
        compute(a + 2, 2)
        s_start(a + 2, 2)
        return carry

    lax.fori_loop(0, _I2, body, 0)

    last = 3 * _I2
    g_wait(last, 0)
    compute(last, 0)
    s_start(last, 0)
    s_wait(last - 2, 1)
    s_wait(last - 1, 2)
    s_wait(last, 0)

    plsc.subcore_barrier()

    @pl.when(s < 15)
    def _():
        pltpu.sync_copy(accum.at[pl.ds(s * 640, 640)],
                        out.at[c, pl.ds(s * 640, 640)])

    @pl.when(s == 15)
    def _():
        pltpu.sync_copy(accum.at[pl.ds(9600, 400)],
                        out.at[c, pl.ds(9600, 400)])


def _sc_conv2(srcr, dstr, txw, as2, ad2):
    mesh = plsc.VectorSubcoreMesh(core_axis_name="c", subcore_axis_name="s",
                                  num_cores=NC, num_subcores=NS)
    f = functools.partial(
        pl.kernel,
        out_type=jax.ShapeDtypeStruct((NC, N, 32), F32),
        mesh=mesh,
        compiler_params=pltpu.CompilerParams(use_tc_tiling_on_sc=False, needs_layout_passes=False),
        scratch_types=[
            pltpu.VMEM((_NCH2, _C2), I32),
            pltpu.VMEM((_NCH2, _C2), I32),
            pltpu.VMEM((_C2, 16), F32),
            pltpu.VMEM((_C2, 16), F32),
            pltpu.VMEM((_C2, 16), F32),
            pltpu.VMEM((_C2, 32), F32),
            pltpu.VMEM((_C2, 32), F32),
            pltpu.VMEM((_C2, 32), F32),
            pltpu.VMEM((N,), F32),
            pltpu.VMEM((N,), F32),
            pltpu.VMEM((40, 32), F32),
            pltpu.VMEM_SHARED((N, 32), F32),
            pltpu.SemaphoreType.DMA,
            pltpu.SemaphoreType.DMA,
            pltpu.SemaphoreType.DMA,
            pltpu.SemaphoreType.DMA,
            pltpu.SemaphoreType.DMA,
            pltpu.SemaphoreType.DMA,
            pltpu.SemaphoreType.DMA,
        ],
    )(_sc_conv2_body)
    return f(srcr, dstr, txw, as2, ad2)


def _sc_iconv_body(srcr, dstr, t3, dinvh, out,
                   sidx, didx, sb0, sb1, sb2, dinvv, zbuf, accum,
                   gs0, gs1, gs2, ss0, ss1, ss2, zsem):
    c = lax.axis_index("c")
    s = lax.axis_index("s")
    wid = s * NC + c
    sbufs = (sb0, sb1, sb2)
    gsems = (gs0, gs1, gs2)
    ssems = (ss0, ss1, ss2)

    pltpu.sync_copy(srcr.at[wid], sidx)
    pltpu.sync_copy(dstr.at[wid], didx)
    pltpu.sync_copy(dinvh, dinvv)

    z16 = jnp.zeros((16,), F32)
    for r in range(40):
        zbuf[r, pl.ds(0, 16)] = z16

    @pl.when(s < 15)
    def _():
        for j in range(16):
            pltpu.async_copy(zbuf, accum.at[pl.ds(s * 640 + j * 40, 40)], zsem)
        for j in range(16):
            pltpu.make_async_copy(zbuf, accum.at[pl.ds(s * 640 + j * 40, 40)], zsem).wait()

    @pl.when(s == 15)
    def _():
        for j in range(10):
            pltpu.async_copy(zbuf, accum.at[pl.ds(9600 + j * 40, 40)], zsem)
        for j in range(10):
            pltpu.make_async_copy(zbuf, accum.at[pl.ds(9600 + j * 40, 40)], zsem).wait()

    def g_start(ci, k):
        pltpu.async_copy(t3.at[sidx.at[ci]], sbufs[k], gsems[k])

    def g_wait(ci, k):
        pltpu.make_async_copy(t3.at[sidx.at[ci]], sbufs[k], gsems[k]).wait()

    def s_start(ci, k):
        pltpu.async_copy(sbufs[k], accum.at[didx.at[ci]], ssems[k], add=True)

    def s_wait(ci, k):
        pltpu.make_async_copy(sbufs[k], accum.at[didx.at[ci]], ssems[k]).wait()

    iota = _iota16()

    def compute(ci, k):
        sb = sbufs[k]
        ci16 = jnp.full((16,), ci, dtype=I32)
        for grp in range(_C2 // 16):
            col = iota + grp * 16
            d16 = plsc.load_gather(didx, [ci16, col])
            dd = plsc.load_gather(dinvv, [d16])
            for e in range(16):
                r = grp * 16 + e
                sp = _take16(dd, e)
                sb[r, pl.ds(0, 16)] = sb[r, pl.ds(0, 16)] * sp

    g_start(0, 0)
    plsc.subcore_barrier()

    def body(i, carry):
        a = 3 * i

        @pl.when(i > 0)
        def _():
            s_wait(a - 2, 1)

        g_start(a + 1, 1)
        g_wait(a, 0)
        compute(a, 0)
        s_start(a, 0)

        @pl.when(i > 0)
        def _():
            s_wait(a - 1, 2)

        g_start(a + 2, 2)
        g_wait(a + 1, 1)
        compute(a + 1, 1)
        s_start(a + 1, 1)

        s_wait(a, 0)
        g_start(a + 3, 0)
        g_wait(a + 2, 2)
        compute(a + 2, 2)
        s_start(a + 2, 2)
        return carry

    lax.fori_loop(0, _I2, body, 0)

    last = 3 * _I2
    g_wait(last, 0)
    compute(last, 0)
    s_start(last, 0)
    s_wait(last - 2, 1)
    s_wait(last - 1, 2)
    s_wait(last, 0)

    plsc.subcore_barrier()

    @pl.when(s < 15)
    def _():
        pltpu.sync_copy(accum.at[pl.ds(s * 640, 640)],
                        out.at[c, pl.ds(s * 640, 640)])

    @pl.when(s == 15)
    def _():
        pltpu.sync_copy(accum.at[pl.ds(9600, 400)],
                        out.at[c, pl.ds(9600, 400)])


def _sc_iconv(srcr, dstr, t3, dinv):
    mesh = plsc.VectorSubcoreMesh(core_axis_name="c", subcore_axis_name="s",
                                  num_cores=NC, num_subcores=NS)
    f = functools.partial(
        pl.kernel,
        out_type=jax.ShapeDtypeStruct((NC, N, 16), F32),
        mesh=mesh,
        compiler_params=pltpu.CompilerParams(use_tc_tiling_on_sc=False, needs_layout_passes=False),
        scratch_types=[
            pltpu.VMEM((_NCH2, _C2), I32),
            pltpu.VMEM((_NCH2, _C2), I32),
            pltpu.VMEM((_C2, 16), F32),
            pltpu.VMEM((_C2, 16), F32),
            pltpu.VMEM((_C2, 16), F32),
            pltpu.VMEM((N,), F32),
            pltpu.VMEM((40, 16), F32),
            pltpu.VMEM_SHARED((N, 16), F32),
            pltpu.SemaphoreType.DMA,
            pltpu.SemaphoreType.DMA,
            pltpu.SemaphoreType.DMA,
            pltpu.SemaphoreType.DMA,
            pltpu.SemaphoreType.DMA,
            pltpu.SemaphoreType.DMA,
            pltpu.SemaphoreType.DMA,
        ],
    )(_sc_iconv_body)
    return f(srcr, dstr, t3, dinv)


_B = 1000


def _tc1_body(x_ref, w1_ref, aa_ref, tsrc_ref, tdst_ref):
    xw = jnp.dot(x_ref[...], w1_ref[...], preferred_element_type=F32)
    asad = jnp.dot(xw, aa_ref[...], preferred_element_type=F32)
    tsrc_ref[:, 0:128] = xw
    tsrc_ref[:, 128:136] = asad[:, 0:8]
    tsrc_ref[:, 136:144] = jnp.zeros((_B, 8), F32)
    tdst_ref[:, 0:8] = asad[:, 8:16]
    tdst_ref[:, 8:16] = jnp.zeros((_B, 8), F32)


def _tc2_body(p_ref, b1_ref, w2_ref, a2_ref, ex_ref, xw2_ref, as2_ref, ad2_ref):
    acc = p_ref[0] + p_ref[1]
    num = acc[:, 0:128]
    den = acc[:, 128:136]
    den_b = jnp.dot(den, ex_ref[...], preferred_element_type=F32)
    h1 = num / (den_b + 1e-30) + b1_ref[...]
    h1 = jnp.where(h1 > 0, h1, jnp.exp(jnp.minimum(h1, 0.0)) - 1.0)
    xw2 = jnp.dot(h1, w2_ref[...], preferred_element_type=F32)
    asad2 = jnp.dot(xw2, a2_ref[...], preferred_element_type=F32)
    xw2_ref[...] = xw2
    as2_ref[...] = asad2[:, 0:1]
    ad2_ref[...] = asad2[:, 1:2]


def _tc3_body(p2_ref, b2_ref, wc_ref, s_ref, t3_ref, self_ref, dinv_ref):
    acc = p2_ref[0] + p2_ref[1]
    db = jnp.dot(acc, s_ref[...], preferred_element_type=F32)
    h2 = acc[:, 0:16] / (db[:, 0:16] + 1e-30) + b2_ref[...]
    dinv_b = lax.rsqrt(1.0 + db[:, 16:32])
    hc = jnp.dot(h2, wc_ref[...], preferred_element_type=F32)
    t3_ref[...] = hc * dinv_b
    self_ref[...] = hc * dinv_b * dinv_b
    dinv_ref[...] = dinv_b[:, 0:1]


def _tc4_body(p3_ref, self_ref, bc_ref, out_ref):
    out_ref[...] = p3_ref[0] + p3_ref[1] + self_ref[...] + bc_ref[...]


def kernel(x, edge_index, W1, a_src1, a_dst1, b1, W2, a_src2, a_dst2, b2, Wc, bc):
    src = edge_index[0].astype(I32)
    dst = edge_index[1].astype(I32)
    src_a = src.reshape(NW, _NCH1, _C1)
    dst_a = dst.reshape(NW, _NCH1, _C1)
    src_b = src.reshape(NW, _NCH2, _C2)
    dst_b = dst.reshape(NW, _NCH2, _C2)

    e8 = jnp.eye(8, dtype=F32)
    asrc_m = (a_src1[:, :, None] * e8[:, None, :]).reshape(128, 8)
    adst_m = (a_dst1[:, :, None] * e8[:, None, :]).reshape(128, 8)
    aa = jnp.concatenate([asrc_m, adst_m], axis=1)
    ex8 = jnp.repeat(e8, 16, axis=1)
    a2 = jnp.concatenate([a_src2.T, a_dst2.T], axis=1)
    smat = jnp.zeros((32, 32), F32).at[16, 0:16].set(1.0).at[17, 16:32].set(1.0)
    b1r = b1.reshape(1, 128)
    b2r = b2.reshape(1, 16)
    bcr = bc.reshape(1, 16)

    grid = (N // _B,)

    tsrc1, tdst1 = pl.pallas_call(
        _tc1_body,
        grid=grid,
        in_specs=[
            pl.BlockSpec((_B, 128), lambda i: (i, 0)),
            pl.BlockSpec((128, 128), lambda i: (0, 0)),
            pl.BlockSpec((128, 16), lambda i: (0, 0)),
        ],
        out_specs=[
            pl.BlockSpec((_B, 144), lambda i: (i, 0)),
            pl.BlockSpec((_B, 16), lambda i: (i, 0)),
        ],
        out_shape=[
            jax.ShapeDtypeStruct((N, 144), F32),
            jax.ShapeDtypeStruct((N, 16), F32),
        ],
    )(x, W1, aa)

    p1 = _sc_conv1(src_a, dst_a, tsrc1, tdst1)

    xw2, as2, ad2 = pl.pallas_call(
        _tc2_body,
        grid=grid,
        in_specs=[
            pl.BlockSpec((NC, _B, 144), lambda i: (0, i, 0)),
            pl.BlockSpec((1, 128), lambda i: (0, 0)),
            pl.BlockSpec((128, 16), lambda i: (0, 0)),
            pl.BlockSpec((16, 2), lambda i: (0, 0)),
            pl.BlockSpec((8, 128), lambda i: (0, 0)),
        ],
        out_specs=[
            pl.BlockSpec((_B, 16), lambda i: (i, 0)),
            pl.BlockSpec((_B, 1), lambda i: (i, 0)),
            pl.BlockSpec((_B, 1), lambda i: (i, 0)),
        ],
        out_shape=[
            jax.ShapeDtypeStruct((N, 16), F32),
            jax.ShapeDtypeStruct((N, 1), F32),
            jax.ShapeDtypeStruct((N, 1), F32),
        ],
    )(p1, b1r, W2, a2, ex8)

    p2 = _sc_conv2(src_b, dst_b, xw2, as2.reshape(N), ad2.reshape(N))

    t3, selfterm, dinv = pl.pallas_call(
        _tc3_body,
        grid=grid,
        in_specs=[
            pl.BlockSpec((NC, _B, 32), lambda i: (0, i, 0)),
            pl.BlockSpec((1, 16), lambda i: (0, 0)),
            pl.BlockSpec((16, 16), lambda i: (0, 0)),
            pl.BlockSpec((32, 32), lambda i: (0, 0)),
        ],
        out_specs=[
            pl.BlockSpec((_B, 16), lambda i: (i, 0)),
            pl.BlockSpec((_B, 16), lambda i: (i, 0)),
            pl.BlockSpec((_B, 1), lambda i: (i, 0)),
        ],
        out_shape=[
            jax.ShapeDtypeStruct((N, 16), F32),
            jax.ShapeDtypeStruct((N, 16), F32),
            jax.ShapeDtypeStruct((N, 1), F32),
        ],
    )(p2, b2r, Wc, smat)

    p3 = _sc_iconv(src_b, dst_b, t3, dinv.reshape(N))

    out = pl.pallas_call(
        _tc4_body,
        grid=grid,
        in_specs=[
            pl.BlockSpec((NC, _B, 16), lambda i: (0, i, 0)),
            pl.BlockSpec((_B, 16), lambda i: (i, 0)),
            pl.BlockSpec((1, 16), lambda i: (0, 0)),
        ],
        out_specs=pl.BlockSpec((_B, 16), lambda i: (i, 0)),
        out_shape=jax.ShapeDtypeStruct((N, 16), F32),
    )(p3, selfterm, bcr)
    return out


# speedup vs baseline: 1.2110x; 1.2110x over previous
import functools

import jax
import jax.numpy as jnp
from jax import lax
from jax.experimental import pallas as pl
from jax.experimental.pallas import tpu as pltpu
from jax.experimental.pallas import tpu_sc as plsc

N = 10000
E = 320000
NC = 2
NS = 16
NW = NC * NS
EW = E // NW
ROWS_PER_TILE = N // NS

F32 = jnp.float32
I32 = jnp.int32


_TAKE_DNUMS = lax.GatherDimensionNumbers(
    offset_dims=(), collapsed_slice_dims=(0,), start_index_map=(0,))


def _take16(v, lane):
    idx = jnp.full((16, 1), lane, dtype=I32)
    return lax.gather(v, idx, _TAKE_DNUMS, slice_sizes=(1,),
                      mode=lax.GatherScatterMode.PROMISE_IN_BOUNDS)


def _iota16():
    return lax.iota(I32, 16)


_C1 = 80
_NCH1 = EW // _C1
_I1 = 41


def _sc_conv1_body(srcr, dstr, tsrc, tdst, out,
                   sidxb, didxb, sb0, sb1, sb2, db0, db1, db2, accum,
                   gs0, gs1, gs2, ds0, ds1, ds2, ss0, ss1, ss2,
                   is0, is1, is2, is3, is4, is5, zsem):
    c = lax.axis_index("c")
    s = lax.axis_index("s")
    wid = s * NC + c
    sbufs = (sb0, sb1, sb2)
    dbufs = (db0, db1, db2)
    gsems = (gs0, gs1, gs2)
    dsems = (ds0, ds1, ds2)
    ssems = (ss0, ss1, ss2)
    isems = (is0, is1, is2, is3, is4, is5)

    def i_start(ci, j):
        pltpu.async_copy(srcr.at[wid, ci], sidxb.at[j], isems[j])
        pltpu.async_copy(dstr.at[wid, ci], didxb.at[j], isems[j])

    def i_wait(ci, j):
        pltpu.make_async_copy(srcr.at[wid, ci], sidxb.at[j], isems[j]).wait()
        pltpu.make_async_copy(dstr.at[wid, ci], didxb.at[j], isems[j]).wait()

    def g_start(ci, k, j):
        pltpu.async_copy(tsrc.at[sidxb.at[j]], sbufs[k], gsems[k])
        pltpu.async_copy(tdst.at[didxb.at[j]], dbufs[k], dsems[k])

    def g_wait(ci, k, j):
        pltpu.make_async_copy(tsrc.at[sidxb.at[j]], sbufs[k], gsems[k]).wait()
        pltpu.make_async_copy(tdst.at[didxb.at[j]], dbufs[k], dsems[k]).wait()

    def s_start(ci, k, j):
        pltpu.async_copy(sbufs[k], accum.at[didxb.at[j]], ssems[k], add=True)

    def s_wait(ci, k, j):
        pltpu.make_async_copy(sbufs[k], accum.at[didxb.at[j]], ssems[k]).wait()

    def compute(k):
        sb = sbufs[k]
        db = dbufs[k]

        @plsc.parallel_loop(0, _C1, unroll=2)
        def _(e):
            a_s = sb[e, pl.ds(128, 16)]
            ad = db[e, pl.ds(0, 16)]
            t = a_s + ad
            w = jnp.exp(jnp.maximum(t, 0.2 * t))
            sb[e, pl.ds(128, 16)] = w
            for h in range(8):
                sp = _take16(w, h)
                sb[e, pl.ds(h * 16, 16)] = sb[e, pl.ds(h * 16, 16)] * sp

    i_start(0, 0)
    i_start(1, 1)
    i_start(2, 2)
    z16 = jnp.zeros((16,), F32)
    for e in range(_C1):
        for l in range(9):
            sb2[e, pl.ds(l * 16, 16)] = z16

    @pl.when(s < 15)
    def _():
        for j in range(8):
            pltpu.async_copy(sb2, accum.at[pl.ds(s * 640 + j * 80, 80)], zsem)
        for j in range(8):
            pltpu.make_async_copy(sb2, accum.at[pl.ds(s * 640 + j * 80, 80)], zsem).wait()

    @pl.when(s == 15)
    def _():
        for j in range(5):
            pltpu.async_copy(sb2, accum.at[pl.ds(9600 + j * 80, 80)], zsem)
        for j in range(5):
            pltpu.make_async_copy(sb2, accum.at[pl.ds(9600 + j * 80, 80)], zsem).wait()

    i_wait(0, 0)
    g_start(0, 0, 0)
    plsc.subcore_barrier()

    def do_third(a, i, S, P):
        i_start(a + 3, P[0])

        @pl.when(i > 0)
        def _():
            s_wait(a - 2, 1, P[1])

        i_start(a + 4, P[1])
        i_wait(a + 1, S[1])
        g_start(a + 1, 1, S[1])
        g_wait(a, 0, S[0])
        compute(0)
        s_start(a, 0, S[0])

        @pl.when(i > 0)
        def _():
            s_wait(a - 1, 2, P[2])

        @pl.when(i < _I1 - 1)
        def _():
            i_start(a + 5, P[2])

        i_wait(a + 2, S[2])
        g_start(a + 2, 2, S[2])
        g_wait(a + 1, 1, S[1])
        compute(1)
        s_start(a + 1, 1, S[1])
        s_wait(a, 0, S[0])
        i_wait(a + 3, P[0])
        g_start(a + 3, 0, P[0])
        g_wait(a + 2, 2, S[2])
        compute(2)
        s_start(a + 2, 2, S[2])

    def body(i, carry):
        a = 3 * i
        par = lax.rem(i, 2)

        @pl.when(par == 0)
        def _():
            do_third(a, i, (0, 1, 2), (3, 4, 5))

        @pl.when(par == 1)
        def _():
            do_third(a, i, (3, 4, 5), (0, 1, 2))

        return carry

    lax.fori_loop(0, _I1, body, 0)

    s_wait(121, 1, 1)
    i_wait(124, 4)
    g_start(124, 1, 4)
    g_wait(123, 0, 3)
    compute(0)
    s_start(123, 0, 3)
    g_wait(124, 1, 4)
    compute(1)
    s_start(124, 1, 4)
    s_wait(122, 2, 2)
    s_wait(123, 0, 3)
    s_wait(124, 1, 4)

    plsc.subcore_barrier()

    @pl.when(s < 15)
    def _():
        pltpu.sync_copy(accum.at[pl.ds(s * 640, 640)],
                        out.at[c, pl.ds(s * 640, 640)])

    @pl.when(s == 15)
    def _():
        pltpu.sync_copy(accum.at[pl.ds(9600, 400)],
                        out.at[c, pl.ds(9600, 400)])


def _sc_conv1(srcr, dstr, tsrc, tdst):
    mesh = plsc.VectorSubcoreMesh(core_axis_name="c", subcore_axis_name="s",
                                  num_cores=NC, num_subcores=NS)
    f = functools.partial(
        pl.kernel,
        out_type=jax.ShapeDtypeStruct((NC, N, 144), F32),
        mesh=mesh,
        compiler_params=pltpu.CompilerParams(use_tc_tiling_on_sc=False, needs_layout_passes=False),
        scratch_types=[
            pltpu.VMEM((6, _C1), I32),
            pltpu.VMEM((6, _C1), I32),
            pltpu.VMEM((_C1, 144), F32),
            pltpu.VMEM((_C1, 144), F32),
            pltpu.VMEM((_C1, 144), F32),
            pltpu.VMEM((_C1, 16), F32),
            pltpu.VMEM((_C1, 16), F32),
            pltpu.VMEM((_C1, 16), F32),
            pltpu.VMEM_SHARED((N, 144), F32),
            pltpu.SemaphoreType.DMA,
            pltpu.SemaphoreType.DMA,
            pltpu.SemaphoreType.DMA,
            pltpu.SemaphoreType.DMA,
            pltpu.SemaphoreType.DMA,
            pltpu.SemaphoreType.DMA,
            pltpu.SemaphoreType.DMA,
            pltpu.SemaphoreType.DMA,
            pltpu.SemaphoreType.DMA,
            pltpu.SemaphoreType.DMA,
            pltpu.SemaphoreType.DMA,
            pltpu.SemaphoreType.DMA,
            pltpu.SemaphoreType.DMA,
            pltpu.SemaphoreType.DMA,
            pltpu.SemaphoreType.DMA,
            pltpu.SemaphoreType.DMA,
        ],
    )(_sc_conv1_body)
    return f(srcr, dstr, tsrc, tdst)


_C2 = 400
_NCH2 = EW // _C2
_I2 = 8


def _sc_conv2_body(srcr, dstr, txw, as2h, ad2h, out,
                   sidx, didx, sb0, sb1, sb2, mb0, mb1, mb2, asv, adv, zbuf, accum,
                   gs0, gs1, gs2, ss0, ss1, ss2, zsem):
    c = lax.axis_index("c")
    s = lax.axis_index("s")
    wid = s * NC + c
    sbufs = (sb0, sb1, sb2)
    mbufs = (mb0, mb1, mb2)
    gsems = (gs0, gs1, gs2)
    ssems = (ss0, ss1, ss2)

    pltpu.sync_copy(srcr.at[wid], sidx)
    pltpu.sync_copy(dstr.at[wid], didx)
    pltpu.sync_copy(as2h, asv)
    pltpu.sync_copy(ad2h, adv)

    z16 = jnp.zeros((16,), F32)
    for r in range(40):
        for l in range(2):
            zbuf[r, pl.ds(l * 16, 16)] = z16

    @pl.when(s < 15)
    def _():
        for j in range(16):
            pltpu.async_copy(zbuf, accum.at[pl.ds(s * 640 + j * 40, 40)], zsem)
        for j in range(16):
            pltpu.make_async_copy(zbuf, accum.at[pl.ds(s * 640 + j * 40, 40)], zsem).wait()

    @pl.when(s == 15)
    def _():
        for j in range(10):
            pltpu.async_copy(zbuf, accum.at[pl.ds(9600 + j * 40, 40)], zsem)
        for j in range(10):
            pltpu.make_async_copy(zbuf, accum.at[pl.ds(9600 + j * 40, 40)], zsem).wait()

    def g_start(ci, k):
        pltpu.async_copy(txw.at[sidx.at[ci]], sbufs[k], gsems[k])

    def g_wait(ci, k):
        pltpu.make_async_copy(txw.at[sidx.at[ci]], sbufs[k], gsems[k]).wait()

    def s_start(ci, k):
        pltpu.async_copy(mbufs[k], accum.at[didx.at[ci]], ssems[k], add=True)

    def s_wait(ci, k):
        pltpu.make_async_copy(mbufs[k], accum.at[didx.at[ci]], ssems[k]).wait()

    iota = _iota16()
    oh0 = jnp.where(iota == 0, 1.0, 0.0).astype(F32)
    oh1 = jnp.where(iota == 1, 1.0, 0.0).astype(F32)

    def compute(ci, k):
        sb = sbufs[k]
        mb = mbufs[k]
        ci16 = jnp.full((16,), ci, dtype=I32)
        for grp in range(_C2 // 16):
            col = iota + grp * 16
            s16 = plsc.load_gather(sidx, [ci16, col])
            d16 = plsc.load_gather(didx, [ci16, col])
            a_s = plsc.load_gather(asv, [s16])
            a_d = plsc.load_gather(adv, [d16])
            t = a_s + a_d
            w = jnp.exp(jnp.maximum(t, 0.2 * t))
            for e in range(16):
                r = grp * 16 + e
                sp = _take16(w, e)
                mb[r, pl.ds(0, 16)] = sb[r, pl.ds(0, 16)] * sp
                mb[r, pl.ds(16, 16)] = sp * oh0 + oh1

    g_start(0, 0)
    plsc.subcore_barrier()

    def body(i, carry):
        a = 3 * i

        @pl.when(i > 0)
        def _():
            s_wait(a - 2, 1)

        g_start(a + 1, 1)
        g_wait(a, 0)
        compute(a, 0)
        s_start(a, 0)

        @pl.when(i > 0)
        def _():
            s_wait(a - 1, 2)

        g_start(a + 2, 2)
        g_wait(a + 1, 1)
        compute(a + 1, 1)
        s_start(a + 1, 1)

        s_wait(a, 0)
        g_start(a + 3, 0)
        g_wait(a + 2, 2)
        compute(a + 2, 2)
        s_start(a + 2, 2)
        return carry

    lax.fori_loop(0, _I2, body, 0)

    last = 3 * _I2
    g_wait(last, 0)
    compute(last, 0)
    s_start(last, 0)
    s_wait(last - 2, 1)
    s_wait(last - 1, 2)
    s_wait(last, 0)

    plsc.subcore_barrier()

    @pl.when(s < 15)
    def _():
        pltpu.sync_copy(accum.at[pl.ds(s * 640, 640)],
                        out.at[c, pl.ds(s * 640, 640)])

    @pl.when(s == 15)
    def _():
        pltpu.sync_copy(accum.at[pl.ds(9600, 400)],
                        out.at[c, pl.ds(9600, 400)])


def _sc_conv2(srcr, dstr, txw, as2, ad2):
    mesh = plsc.VectorSubcoreMesh(core_axis_name="c", subcore_axis_name="s",
                                  num_cores=NC, num_subcores=NS)
    f = functools.partial(
        pl.kernel,
        out_type=jax.ShapeDtypeStruct((NC, N, 32), F32),
        mesh=mesh,
        compiler_params=pltpu.CompilerParams(use_tc_tiling_on_sc=False, needs_layout_passes=False),
        scratch_types=[
            pltpu.VMEM((_NCH2, _C2), I32),
            pltpu.VMEM((_NCH2, _C2), I32),
            pltpu.VMEM((_C2, 16), F32),
            pltpu.VMEM((_C2, 16), F32),
            pltpu.VMEM((_C2, 16), F32),
            pltpu.VMEM((_C2, 32), F32),
            pltpu.VMEM((_C2, 32), F32),
            pltpu.VMEM((_C2, 32), F32),
            pltpu.VMEM((N,), F32),
            pltpu.VMEM((N,), F32),
            pltpu.VMEM((40, 32), F32),
            pltpu.VMEM_SHARED((N, 32), F32),
            pltpu.SemaphoreType.DMA,
            pltpu.SemaphoreType.DMA,
            pltpu.SemaphoreType.DMA,
            pltpu.SemaphoreType.DMA,
            pltpu.SemaphoreType.DMA,
            pltpu.SemaphoreType.DMA,
            pltpu.SemaphoreType.DMA,
        ],
    )(_sc_conv2_body)
    return f(srcr, dstr, txw, as2, ad2)


def _sc_iconv_body(srcr, dstr, t3, dinvh, out,
                   sidx, didx, sb0, sb1, sb2, dinvv, zbuf, accum,
                   gs0, gs1, gs2, ss0, ss1, ss2, zsem):
    c = lax.axis_index("c")
    s = lax.axis_index("s")
    wid = s * NC + c
    sbufs = (sb0, sb1, sb2)
    gsems = (gs0, gs1, gs2)
    ssems = (ss0, ss1, ss2)

    pltpu.sync_copy(srcr.at[wid], sidx)
    pltpu.sync_copy(dstr.at[wid], didx)
    pltpu.sync_copy(dinvh, dinvv)

    z16 = jnp.zeros((16,), F32)
    for r in range(40):
        zbuf[r, pl.ds(0, 16)] = z16

    @pl.when(s < 15)
    def _():
        for j in range(16):
            pltpu.async_copy(zbuf, accum.at[pl.ds(s * 640 + j * 40, 40)], zsem)
        for j in range(16):
            pltpu.make_async_copy(zbuf, accum.at[pl.ds(s * 640 + j * 40, 40)], zsem).wait()

    @pl.when(s == 15)
    def _():
        for j in range(10):
            pltpu.async_copy(zbuf, accum.at[pl.ds(9600 + j * 40, 40)], zsem)
        for j in range(10):
            pltpu.make_async_copy(zbuf, accum.at[pl.ds(9600 + j * 40, 40)], zsem).wait()

    def g_start(ci, k):
        pltpu.async_copy(t3.at[sidx.at[ci]], sbufs[k], gsems[k])

    def g_wait(ci, k):
        pltpu.make_async_copy(t3.at[sidx.at[ci]], sbufs[k], gsems[k]).wait()

    def s_start(ci, k):
        pltpu.async_copy(sbufs[k], accum.at[didx.at[ci]], ssems[k], add=True)

    def s_wait(ci, k):
        pltpu.make_async_copy(sbufs[k], accum.at[didx.at[ci]], ssems[k]).wait()

    iota = _iota16()

    def compute(ci, k):
        sb = sbufs[k]
        ci16 = jnp.full((16,), ci, dtype=I32)
        for grp in range(_C2 // 16):
            col = iota + grp * 16
            d16 = plsc.load_gather(didx, [ci16, col])
            dd = plsc.load_gather(dinvv, [d16])
            for e in range(16):
                r = grp * 16 + e
                sp = _take16(dd, e)
                sb[r, pl.ds(0, 16)] = sb[r, pl.ds(0, 16)] * sp

    g_start(0, 0)
    plsc.subcore_barrier()

    def body(i, carry):
        a = 3 * i

        @pl.when(i > 0)
        def _():
            s_wait(a - 2, 1)

        g_start(a + 1, 1)
        g_wait(a, 0)
        compute(a, 0)
        s_start(a, 0)

        @pl.when(i > 0)
        def _():
            s_wait(a - 1, 2)

        g_start(a + 2, 2)
        g_wait(a + 1, 1)
        compute(a + 1, 1)
        s_start(a + 1, 1)

        s_wait(a, 0)
        g_start(a + 3, 0)
        g_wait(a + 2, 2)
        compute(a + 2, 2)
        s_start(a + 2, 2)
        return carry

    lax.fori_loop(0, _I2, body, 0)

    last = 3 * _I2
    g_wait(last, 0)
    compute(last, 0)
    s_start(last, 0)
    s_wait(last - 2, 1)
    s_wait(last - 1, 2)
    s_wait(last, 0)

    plsc.subcore_barrier()

    @pl.when(s < 15)
    def _():
        pltpu.sync_copy(accum.at[pl.ds(s * 640, 640)],
                        out.at[c, pl.ds(s * 640, 640)])

    @pl.when(s == 15)
    def _():
        pltpu.sync_copy(accum.at[pl.ds(9600, 400)],
                        out.at[c, pl.ds(9600, 400)])


def _sc_iconv(srcr, dstr, t3, dinv):
    mesh = plsc.VectorSubcoreMesh(core_axis_name="c", subcore_axis_name="s",
                                  num_cores=NC, num_subcores=NS)
    f = functools.partial(
        pl.kernel,
        out_type=jax.ShapeDtypeStruct((NC, N, 16), F32),
        mesh=mesh,
        compiler_params=pltpu.CompilerParams(use_tc_tiling_on_sc=False, needs_layout_passes=False),
        scratch_types=[
            pltpu.VMEM((_NCH2, _C2), I32),
            pltpu.VMEM((_NCH2, _C2), I32),
            pltpu.VMEM((_C2, 16), F32),
            pltpu.VMEM((_C2, 16), F32),
            pltpu.VMEM((_C2, 16), F32),
            pltpu.VMEM((N,), F32),
            pltpu.VMEM((40, 16), F32),
            pltpu.VMEM_SHARED((N, 16), F32),
            pltpu.SemaphoreType.DMA,
            pltpu.SemaphoreType.DMA,
            pltpu.SemaphoreType.DMA,
            pltpu.SemaphoreType.DMA,
            pltpu.SemaphoreType.DMA,
            pltpu.SemaphoreType.DMA,
            pltpu.SemaphoreType.DMA,
        ],
    )(_sc_iconv_body)
    return f(srcr, dstr, t3, dinv)


_B = 1000


def _tc1_body(x_ref, w1_ref, aa_ref, tsrc_ref, tdst_ref):
    xw = jnp.dot(x_ref[...], w1_ref[...], preferred_element_type=F32)
    asad = jnp.dot(xw, aa_ref[...], preferred_element_type=F32)
    tsrc_ref[:, 0:128] = xw
    tsrc_ref[:, 128:136] = asad[:, 0:8]
    tsrc_ref[:, 136:144] = jnp.zeros((_B, 8), F32)
    tdst_ref[:, 0:8] = asad[:, 8:16]
    tdst_ref[:, 8:16] = jnp.zeros((_B, 8), F32)


def _tc2_body(p_ref, b1_ref, w2_ref, a2_ref, ex_ref, xw2_ref, as2_ref, ad2_ref):
    acc = p_ref[0] + p_ref[1]
    num = acc[:, 0:128]
    den = acc[:, 128:136]
    den_b = jnp.dot(den, ex_ref[...], preferred_element_type=F32)
    h1 = num / (den_b + 1e-30) + b1_ref[...]
    h1 = jnp.where(h1 > 0, h1, jnp.exp(jnp.minimum(h1, 0.0)) - 1.0)
    xw2 = jnp.dot(h1, w2_ref[...], preferred_element_type=F32)
    asad2 = jnp.dot(xw2, a2_ref[...], preferred_element_type=F32)
    xw2_ref[...] = xw2
    as2_ref[...] = asad2[:, 0:1]
    ad2_ref[...] = asad2[:, 1:2]


def _tc3_body(p2_ref, b2_ref, wc_ref, s_ref, t3_ref, self_ref, dinv_ref):
    acc = p2_ref[0] + p2_ref[1]
    db = jnp.dot(acc, s_ref[...], preferred_element_type=F32)
    h2 = acc[:, 0:16] / (db[:, 0:16] + 1e-30) + b2_ref[...]
    dinv_b = lax.rsqrt(1.0 + db[:, 16:32])
    hc = jnp.dot(h2, wc_ref[...], preferred_element_type=F32)
    t3_ref[...] = hc * dinv_b
    self_ref[...] = hc * dinv_b * dinv_b
    dinv_ref[...] = dinv_b[:, 0:1]


def _tc4_body(p3_ref, self_ref, bc_ref, out_ref):
    out_ref[...] = p3_ref[0] + p3_ref[1] + self_ref[...] + bc_ref[...]


def kernel(x, edge_index, W1, a_src1, a_dst1, b1, W2, a_src2, a_dst2, b2, Wc, bc):
    src = edge_index[0].astype(I32)
    dst = edge_index[1].astype(I32)
    src_a = src.reshape(NW, _NCH1, _C1)
    dst_a = dst.reshape(NW, _NCH1, _C1)
    src_b = src.reshape(NW, _NCH2, _C2)
    dst_b = dst.reshape(NW, _NCH2, _C2)

    e8 = jnp.eye(8, dtype=F32)
    asrc_m = (a_src1[:, :, None] * e8[:, None, :]).reshape(128, 8)
    adst_m = (a_dst1[:, :, None] * e8[:, None, :]).reshape(128, 8)
    aa = jnp.concatenate([asrc_m, adst_m], axis=1)
    ex8 = jnp.repeat(e8, 16, axis=1)
    a2 = jnp.concatenate([a_src2.T, a_dst2.T], axis=1)
    smat = jnp.zeros((32, 32), F32).at[16, 0:16].set(1.0).at[17, 16:32].set(1.0)
    b1r = b1.reshape(1, 128)
    b2r = b2.reshape(1, 16)
    bcr = bc.reshape(1, 16)

    grid = (N // _B,)

    tsrc1, tdst1 = pl.pallas_call(
        _tc1_body,
        grid=grid,
        in_specs=[
            pl.BlockSpec((_B, 128), lambda i: (i, 0)),
            pl.BlockSpec((128, 128), lambda i: (0, 0)),
            pl.BlockSpec((128, 16), lambda i: (0, 0)),
        ],
        out_specs=[
            pl.BlockSpec((_B, 144), lambda i: (i, 0)),
            pl.BlockSpec((_B, 16), lambda i: (i, 0)),
        ],
        out_shape=[
            jax.ShapeDtypeStruct((N, 144), F32),
            jax.ShapeDtypeStruct((N, 16), F32),
        ],
    )(x, W1, aa)

    p1 = _sc_conv1(src_a, dst_a, tsrc1, tdst1)

    xw2, as2, ad2 = pl.pallas_call(
        _tc2_body,
        grid=grid,
        in_specs=[
            pl.BlockSpec((NC, _B, 144), lambda i: (0, i, 0)),
            pl.BlockSpec((1, 128), lambda i: (0, 0)),
            pl.BlockSpec((128, 16), lambda i: (0, 0)),
            pl.BlockSpec((16, 2), lambda i: (0, 0)),
            pl.BlockSpec((8, 128), lambda i: (0, 0)),
        ],
        out_specs=[
            pl.BlockSpec((_B, 16), lambda i: (i, 0)),
            pl.BlockSpec((_B, 1), lambda i: (i, 0)),
            pl.BlockSpec((_B, 1), lambda i: (i, 0)),
        ],
        out_shape=[
            jax.ShapeDtypeStruct((N, 16), F32),
            jax.ShapeDtypeStruct((N, 1), F32),
            jax.ShapeDtypeStruct((N, 1), F32),
        ],
    )(p1, b1r, W2, a2, ex8)

    p2 = _sc_conv2(src_b, dst_b, xw2, as2.reshape(N), ad2.reshape(N))

    t3, selfterm, dinv = pl.pallas_call(
        _tc3_body,
        grid=grid,
        in_specs=[
            pl.BlockSpec((NC, _B, 32), lambda i: (0, i, 0)),
            pl.BlockSpec((1, 16), lambda i: (0, 0)),
            pl.BlockSpec((16, 16), lambda i: (0, 0)),
            pl.BlockSpec((32, 32), lambda i: (0, 0)),
        ],
        out_specs=[
            pl.BlockSpec((_B, 16), lambda i: (i, 0)),
            pl.BlockSpec((_B, 16), lambda i: (i, 0)),
            pl.BlockSpec((_B, 1), lambda i: (i, 0)),
        ],
        out_shape=[
            jax.ShapeDtypeStruct((N, 16), F32),
            jax.ShapeDtypeStruct((N, 16), F32),
            jax.ShapeDtypeStruct((N, 1), F32),
        ],
    )(p2, b2r, Wc, smat)

    p3 = _sc_iconv(src_b, dst_b, t3, dinv.reshape(N))

    out = pl.pallas_call(
        _tc4_body,
        grid=grid,
        in_specs=[
            pl.BlockSpec((NC, _B, 16), lambda i: (0, i, 0)),
            pl.BlockSpec((_B, 16), lambda i: (i, 0)),
            pl.BlockSpec((1, 16), lambda i: (0, 0)),
        ],
        out_specs=pl.BlockSpec((_B, 16), lambda i: (i, 0)),
        out_shape=jax.ShapeDtypeStruct((N, 16), F32),
    )(p3, selfterm, bcr)
    return out


# speedup vs baseline: 1.2133x; 1.0020x over previous
import functools

import jax
import jax.numpy as jnp
from jax import lax
from jax.experimental import pallas as pl
from jax.experimental.pallas import tpu as pltpu
from jax.experimental.pallas import tpu_sc as plsc

N = 10000
E = 320000
NC = 2
NS = 16
NW = NC * NS
EW = E // NW
ROWS_PER_TILE = N // NS

F32 = jnp.float32
I32 = jnp.int32


_TAKE_DNUMS = lax.GatherDimensionNumbers(
    offset_dims=(), collapsed_slice_dims=(0,), start_index_map=(0,))


def _take16(v, lane):
    idx = jnp.full((16, 1), lane, dtype=I32)
    return lax.gather(v, idx, _TAKE_DNUMS, slice_sizes=(1,),
                      mode=lax.GatherScatterMode.PROMISE_IN_BOUNDS)


def _iota16():
    return lax.iota(I32, 16)


_C1 = 80
_NCH1 = EW // _C1
_I1 = 41


def _sc_conv1_body(srcr, dstr, tsrc, tdst, out,
                   sidxb, didxb, sb0, sb1, sb2, db0, db1, db2, accum,
                   gs0, gs1, gs2, ds0, ds1, ds2, ss0, ss1, ss2,
                   is0, is1, is2, is3, is4, is5, zsem):
    c = lax.axis_index("c")
    s = lax.axis_index("s")
    wid = s * NC + c
    sbufs = (sb0, sb1, sb2)
    dbufs = (db0, db1, db2)
    gsems = (gs0, gs1, gs2)
    dsems = (ds0, ds1, ds2)
    ssems = (ss0, ss1, ss2)
    isems = (is0, is1, is2, is3, is4, is5)

    def i_start(ci, j):
        pltpu.async_copy(srcr.at[wid, ci], sidxb.at[j], isems[j])
        pltpu.async_copy(dstr.at[wid, ci], didxb.at[j], isems[j])

    def i_wait(ci, j):
        pltpu.make_async_copy(srcr.at[wid, ci], sidxb.at[j], isems[j]).wait()
        pltpu.make_async_copy(dstr.at[wid, ci], didxb.at[j], isems[j]).wait()

    def g_start(ci, k, j):
        pltpu.async_copy(tsrc.at[sidxb.at[j]], sbufs[k], gsems[k])
        pltpu.async_copy(tdst.at[didxb.at[j]], dbufs[k], dsems[k])

    def g_wait(ci, k, j):
        pltpu.make_async_copy(tsrc.at[sidxb.at[j]], sbufs[k], gsems[k]).wait()
        pltpu.make_async_copy(tdst.at[didxb.at[j]], dbufs[k], dsems[k]).wait()

    def s_start(ci, k, j):
        pltpu.async_copy(sbufs[k], accum.at[didxb.at[j]], ssems[k], add=True)

    def s_wait(ci, k, j):
        pltpu.make_async_copy(sbufs[k], accum.at[didxb.at[j]], ssems[k]).wait()

    def compute(k):
        sb = sbufs[k]
        db = dbufs[k]

        @plsc.parallel_loop(0, _C1, unroll=4)
        def _(e):
            a_s = sb[e, pl.ds(128, 16)]
            ad = db[e, pl.ds(0, 16)]
            t = a_s + ad
            w = jnp.exp(jnp.maximum(t, 0.2 * t))
            sb[e, pl.ds(128, 16)] = w
            for h in range(8):
                sp = _take16(w, h)
                sb[e, pl.ds(h * 16, 16)] = sb[e, pl.ds(h * 16, 16)] * sp

    i_start(0, 0)
    i_start(1, 1)
    i_start(2, 2)
    z16 = jnp.zeros((16,), F32)
    for e in range(_C1):
        for l in range(9):
            sb2[e, pl.ds(l * 16, 16)] = z16

    @pl.when(s < 15)
    def _():
        for j in range(8):
            pltpu.async_copy(sb2, accum.at[pl.ds(s * 640 + j * 80, 80)], zsem)
        for j in range(8):
            pltpu.make_async_copy(sb2, accum.at[pl.ds(s * 640 + j * 80, 80)], zsem).wait()

    @pl.when(s == 15)
    def _():
        for j in range(5):
            pltpu.async_copy(sb2, accum.at[pl.ds(9600 + j * 80, 80)], zsem)
        for j in range(5):
            pltpu.make_async_copy(sb2, accum.at[pl.ds(9600 + j * 80, 80)], zsem).wait()

    i_wait(0, 0)
    g_start(0, 0, 0)
    plsc.subcore_barrier()

    def do_third(a, i, S, P):
        i_start(a + 3, P[0])

        @pl.when(i > 0)
        def _():
            s_wait(a - 2, 1, P[1])

        i_start(a + 4, P[1])
        i_wait(a + 1, S[1])
        g_start(a + 1, 1, S[1])
        g_wait(a, 0, S[0])
        compute(0)
        s_start(a, 0, S[0])

        @pl.when(i > 0)
        def _():
            s_wait(a - 1, 2, P[2])

        @pl.when(i < _I1 - 1)
        def _():
            i_start(a + 5, P[2])

        i_wait(a + 2, S[2])
        g_start(a + 2, 2, S[2])
        g_wait(a + 1, 1, S[1])
        compute(1)
        s_start(a + 1, 1, S[1])
        s_wait(a, 0, S[0])
        i_wait(a + 3, P[0])
        g_start(a + 3, 0, P[0])
        g_wait(a + 2, 2, S[2])
        compute(2)
        s_start(a + 2, 2, S[2])

    def body(i, carry):
        a = 3 * i
        par = lax.rem(i, 2)

        @pl.when(par == 0)
        def _():
            do_third(a, i, (0, 1, 2), (3, 4, 5))

        @pl.when(par == 1)
        def _():
            do_third(a, i, (3, 4, 5), (0, 1, 2))

        return carry

    lax.fori_loop(0, _I1, body, 0)

    s_wait(121, 1, 1)
    i_wait(124, 4)
    g_start(124, 1, 4)
    g_wait(123, 0, 3)
    compute(0)
    s_start(123, 0, 3)
    g_wait(124, 1, 4)
    compute(1)
    s_start(124, 1, 4)
    s_wait(122, 2, 2)
    s_wait(123, 0, 3)
    s_wait(124, 1, 4)

    plsc.subcore_barrier()

    @pl.when(s < 15)
    def _():
        pltpu.sync_copy(accum.at[pl.ds(s * 640, 640)],
                        out.at[c, pl.ds(s * 640, 640)])

    @pl.when(s == 15)
    def _():
        pltpu.sync_copy(accum.at[pl.ds(9600, 400)],
                        out.at[c, pl.ds(9600, 400)])


def _sc_conv1(srcr, dstr, tsrc, tdst):
    mesh = plsc.VectorSubcoreMesh(core_axis_name="c", subcore_axis_name="s",
                                  num_cores=NC, num_subcores=NS)
    f = functools.partial(
        pl.kernel,
        out_type=jax.ShapeDtypeStruct((NC, N, 144), F32),
        mesh=mesh,
        compiler_params=pltpu.CompilerParams(use_tc_tiling_on_sc=False, needs_layout_passes=False),
        scratch_types=[
            pltpu.VMEM((6, _C1), I32),
            pltpu.VMEM((6, _C1), I32),
            pltpu.VMEM((_C1, 144), F32),
            pltpu.VMEM((_C1, 144), F32),
            pltpu.VMEM((_C1, 144), F32),
            pltpu.VMEM((_C1, 16), F32),
            pltpu.VMEM((_C1, 16), F32),
            pltpu.VMEM((_C1, 16), F32),
            pltpu.VMEM_SHARED((N, 144), F32),
            pltpu.SemaphoreType.DMA,
            pltpu.SemaphoreType.DMA,
            pltpu.SemaphoreType.DMA,
            pltpu.SemaphoreType.DMA,
            pltpu.SemaphoreType.DMA,
            pltpu.SemaphoreType.DMA,
            pltpu.SemaphoreType.DMA,
            pltpu.SemaphoreType.DMA,
            pltpu.SemaphoreType.DMA,
            pltpu.SemaphoreType.DMA,
            pltpu.SemaphoreType.DMA,
            pltpu.SemaphoreType.DMA,
            pltpu.SemaphoreType.DMA,
            pltpu.SemaphoreType.DMA,
            pltpu.SemaphoreType.DMA,
            pltpu.SemaphoreType.DMA,
        ],
    )(_sc_conv1_body)
    return f(srcr, dstr, tsrc, tdst)


_C2 = 400
_NCH2 = EW // _C2
_I2 = 8


def _sc_conv2_body(srcr, dstr, txw, as2h, ad2h, out,
                   sidx, didx, sb0, sb1, sb2, mb0, mb1, mb2, asv, adv, zbuf, accum,
                   gs0, gs1, gs2, ss0, ss1, ss2, zsem):
    c = lax.axis_index("c")
    s = lax.axis_index("s")
    wid = s * NC + c
    sbufs = (sb0, sb1, sb2)
    mbufs = (mb0, mb1, mb2)
    gsems = (gs0, gs1, gs2)
    ssems = (ss0, ss1, ss2)

    pltpu.sync_copy(srcr.at[wid], sidx)
    pltpu.sync_copy(dstr.at[wid], didx)
    pltpu.sync_copy(as2h, asv)
    pltpu.sync_copy(ad2h, adv)

    z16 = jnp.zeros((16,), F32)
    for r in range(40):
        for l in range(2):
            zbuf[r, pl.ds(l * 16, 16)] = z16

    @pl.when(s < 15)
    def _():
        for j in range(16):
            pltpu.async_copy(zbuf, accum.at[pl.ds(s * 640 + j * 40, 40)], zsem)
        for j in range(16):
            pltpu.make_async_copy(zbuf, accum.at[pl.ds(s * 640 + j * 40, 40)], zsem).wait()

    @pl.when(s == 15)
    def _():
        for j in range(10):
            pltpu.async_copy(zbuf, accum.at[pl.ds(9600 + j * 40, 40)], zsem)
        for j in range(10):
            pltpu.make_async_copy(zbuf, accum.at[pl.ds(9600 + j * 40, 40)], zsem).wait()

    def g_start(ci, k):
        pltpu.async_copy(txw.at[sidx.at[ci]], sbufs[k], gsems[k])

    def g_wait(ci, k):
        pltpu.make_async_copy(txw.at[sidx.at[ci]], sbufs[k], gsems[k]).wait()

    def s_start(ci, k):
        pltpu.async_copy(mbufs[k], accum.at[didx.at[ci]], ssems[k], add=True)

    def s_wait(ci, k):
        pltpu.make_async_copy(mbufs[k], accum.at[didx.at[ci]], ssems[k]).wait()

    iota = _iota16()
    oh0 = jnp.where(iota == 0, 1.0, 0.0).astype(F32)
    oh1 = jnp.where(iota == 1, 1.0, 0.0).astype(F32)

    def compute(ci, k):
        sb = sbufs[k]
        mb = mbufs[k]
        ci16 = jnp.full((16,), ci, dtype=I32)
        for grp in range(_C2 // 16):
            col = iota + grp * 16
            s16 = plsc.load_gather(sidx, [ci16, col])
            d16 = plsc.load_gather(didx, [ci16, col])
            a_s = plsc.load_gather(asv, [s16])
            a_d = plsc.load_gather(adv, [d16])
            t = a_s + a_d
            w = jnp.exp(jnp.maximum(t, 0.2 * t))
            for e in range(16):
                r = grp * 16 + e
                sp = _take16(w, e)
                mb[r, pl.ds(0, 16)] = sb[r, pl.ds(0, 16)] * sp
                mb[r, pl.ds(16, 16)] = sp * oh0 + oh1

    g_start(0, 0)
    plsc.subcore_barrier()

    def body(i, carry):
        a = 3 * i

        @pl.when(i > 0)
        def _():
            s_wait(a - 2, 1)

        g_start(a + 1, 1)
        g_wait(a, 0)
        compute(a, 0)
        s_start(a, 0)

        @pl.when(i > 0)
        def _():
            s_wait(a - 1, 2)

        g_start(a + 2, 2)
        g_wait(a + 1, 1)
        compute(a + 1, 1)
        s_start(a + 1, 1)

        s_wait(a, 0)
        g_start(a + 3, 0)
        g_wait(a + 2, 2)
        compute(a + 2, 2)
        s_start(a + 2, 2)
        return carry

    lax.fori_loop(0, _I2, body, 0)

    last = 3 * _I2
    g_wait(last, 0)
    compute(last, 0)
    s_start(last, 0)
    s_wait(last - 2, 1)
    s_wait(last - 1, 2)
    s_wait(last, 0)

    plsc.subcore_barrier()

    @pl.when(s < 15)
    def _():
        pltpu.sync_copy(accum.at[pl.ds(s * 640, 640)],
                        out.at[c, pl.ds(s * 640, 640)])

    @pl.when(s == 15)
    def _():
        pltpu.sync_copy(accum.at[pl.ds(9600, 400)],
                        out.at[c, pl.ds(9600, 400)])


def _sc_conv2(srcr, dstr, txw, as2, ad2):
    mesh = plsc.VectorSubcoreMesh(core_axis_name="c", subcore_axis_name="s",
                                  num_cores=NC, num_subcores=NS)
    f = functools.partial(
        pl.kernel,
        out_type=jax.ShapeDtypeStruct((NC, N, 32), F32),
        mesh=mesh,
        compiler_params=pltpu.CompilerParams(use_tc_tiling_on_sc=False, needs_layout_passes=False),
        scratch_types=[
            pltpu.VMEM((_NCH2, _C2), I32),
            pltpu.VMEM((_NCH2, _C2), I32),
            pltpu.VMEM((_C2, 16), F32),
            pltpu.VMEM((_C2, 16), F32),
            pltpu.VMEM((_C2, 16), F32),
            pltpu.VMEM((_C2, 32), F32),
            pltpu.VMEM((_C2, 32), F32),
            pltpu.VMEM((_C2, 32), F32),
            pltpu.VMEM((N,), F32),
            pltpu.VMEM((N,), F32),
            pltpu.VMEM((40, 32), F32),
            pltpu.VMEM_SHARED((N, 32), F32),
            pltpu.SemaphoreType.DMA,
            pltpu.SemaphoreType.DMA,
            pltpu.SemaphoreType.DMA,
            pltpu.SemaphoreType.DMA,
            pltpu.SemaphoreType.DMA,
            pltpu.SemaphoreType.DMA,
            pltpu.SemaphoreType.DMA,
        ],
    )(_sc_conv2_body)
    return f(srcr, dstr, txw, as2, ad2)


def _sc_iconv_body(srcr, dstr, t3, dinvh, out,
                   sidx, didx, sb0, sb1, sb2, dinvv, zbuf, accum,
                   gs0, gs1, gs2, ss0, ss1, ss2, zsem):
    c = lax.axis_index("c")
    s = lax.axis_index("s")
    wid = s * NC + c
    sbufs = (sb0, sb1, sb2)
    gsems = (gs0, gs1, gs2)
    ssems = (ss0, ss1, ss2)

    pltpu.sync_copy(srcr.at[wid], sidx)
    pltpu.sync_copy(dstr.at[wid], didx)
    pltpu.sync_copy(dinvh, dinvv)

    z16 = jnp.zeros((16,), F32)
    for r in range(40):
        zbuf[r, pl.ds(0, 16)] = z16

    @pl.when(s < 15)
    def _():
        for j in range(16):
            pltpu.async_copy(zbuf, accum.at[pl.ds(s * 640 + j * 40, 40)], zsem)
        for j in range(16):
            pltpu.make_async_copy(zbuf, accum.at[pl.ds(s * 640 + j * 40, 40)], zsem).wait()

    @pl.when(s == 15)
    def _():
        for j in range(10):
            pltpu.async_copy(zbuf, accum.at[pl.ds(9600 + j * 40, 40)], zsem)
        for j in range(10):
            pltpu.make_async_copy(zbuf, accum.at[pl.ds(9600 + j * 40, 40)], zsem).wait()

    def g_start(ci, k):
        pltpu.async_copy(t3.at[sidx.at[ci]], sbufs[k], gsems[k])

    def g_wait(ci, k):
        pltpu.make_async_copy(t3.at[sidx.at[ci]], sbufs[k], gsems[k]).wait()

    def s_start(ci, k):
        pltpu.async_copy(sbufs[k], accum.at[didx.at[ci]], ssems[k], add=True)

    def s_wait(ci, k):
        pltpu.make_async_copy(sbufs[k], accum.at[didx.at[ci]], ssems[k]).wait()

    iota = _iota16()

    def compute(ci, k):
        sb = sbufs[k]
        ci16 = jnp.full((16,), ci, dtype=I32)
        for grp in range(_C2 // 16):
            col = iota + grp * 16
            d16 = plsc.load_gather(didx, [ci16, col])
            dd = plsc.load_gather(dinvv, [d16])
            for e in range(16):
                r = grp * 16 + e
                sp = _take16(dd, e)
                sb[r, pl.ds(0, 16)] = sb[r, pl.ds(0, 16)] * sp

    g_start(0, 0)
    plsc.subcore_barrier()

    def body(i, carry):
        a = 3 * i

        @pl.when(i > 0)
        def _():
            s_wait(a - 2, 1)

        g_start(a + 1, 1)
        g_wait(a, 0)
        compute(a, 0)
        s_start(a, 0)

        @pl.when(i > 0)
        def _():
            s_wait(a - 1, 2)

        g_start(a + 2, 2)
        g_wait(a + 1, 1)
        compute(a + 1, 1)
        s_start(a + 1, 1)

        s_wait(a, 0)
        g_start(a + 3, 0)
        g_wait(a + 2, 2)
        compute(a + 2, 2)
        s_start(a + 2, 2)
        return carry

    lax.fori_loop(0, _I2, body, 0)

    last = 3 * _I2
    g_wait(last, 0)
    compute(last, 0)
    s_start(last, 0)
    s_wait(last - 2, 1)
    s_wait(last - 1, 2)
    s_wait(last, 0)

    plsc.subcore_barrier()

    @pl.when(s < 15)
    def _():
        pltpu.sync_copy(accum.at[pl.ds(s * 640, 640)],
                        out.at[c, pl.ds(s * 640, 640)])

    @pl.when(s == 15)
    def _():
        pltpu.sync_copy(accum.at[pl.ds(9600, 400)],
                        out.at[c, pl.ds(9600, 400)])


def _sc_iconv(srcr, dstr, t3, dinv):
    mesh = plsc.VectorSubcoreMesh(core_axis_name="c", subcore_axis_name="s",
                                  num_cores=NC, num_subcores=NS)
    f = functools.partial(
        pl.kernel,
        out_type=jax.ShapeDtypeStruct((NC, N, 16), F32),
        mesh=mesh,
        compiler_params=pltpu.CompilerParams(use_tc_tiling_on_sc=False, needs_layout_passes=False),
        scratch_types=[
            pltpu.VMEM((_NCH2, _C2), I32),
            pltpu.VMEM((_NCH2, _C2), I32),
            pltpu.VMEM((_C2, 16), F32),
            pltpu.VMEM((_C2, 16), F32),
            pltpu.VMEM((_C2, 16), F32),
            pltpu.VMEM((N,), F32),
            pltpu.VMEM((40, 16), F32),
            pltpu.VMEM_SHARED((N, 16), F32),
            pltpu.SemaphoreType.DMA,
            pltpu.SemaphoreType.DMA,
            pltpu.SemaphoreType.DMA,
            pltpu.SemaphoreType.DMA,
            pltpu.SemaphoreType.DMA,
            pltpu.SemaphoreType.DMA,
            pltpu.SemaphoreType.DMA,
        ],
    )(_sc_iconv_body)
    return f(srcr, dstr, t3, dinv)


_B = 1000


def _tc1_body(x_ref, w1_ref, aa_ref, tsrc_ref, tdst_ref):
    xw = jnp.dot(x_ref[...], w1_ref[...], preferred_element_type=F32)
    asad = jnp.dot(xw, aa_ref[...], preferred_element_type=F32)
    tsrc_ref[:, 0:128] = xw
    tsrc_ref[:, 128:136] = asad[:, 0:8]
    tsrc_ref[:, 136:144] = jnp.zeros((_B, 8), F32)
    tdst_ref[:, 0:8] = asad[:, 8:16]
    tdst_ref[:, 8:16] = jnp.zeros((_B, 8), F32)


def _tc2_body(p_ref, b1_ref, w2_ref, a2_ref, ex_ref, xw2_ref, as2_ref, ad2_ref):
    acc = p_ref[0] + p_ref[1]
    num = acc[:, 0:128]
    den = acc[:, 128:136]
    den_b = jnp.dot(den, ex_ref[...], preferred_element_type=F32)
    h1 = num / (den_b + 1e-30) + b1_ref[...]
    h1 = jnp.where(h1 > 0, h1, jnp.exp(jnp.minimum(h1, 0.0)) - 1.0)
    xw2 = jnp.dot(h1, w2_ref[...], preferred_element_type=F32)
    asad2 = jnp.dot(xw2, a2_ref[...], preferred_element_type=F32)
    xw2_ref[...] = xw2
    as2_ref[...] = asad2[:, 0:1]
    ad2_ref[...] = asad2[:, 1:2]


def _tc3_body(p2_ref, b2_ref, wc_ref, s_ref, t3_ref, self_ref, dinv_ref):
    acc = p2_ref[0] + p2_ref[1]
    db = jnp.dot(acc, s_ref[...], preferred_element_type=F32)
    h2 = acc[:, 0:16] / (db[:, 0:16] + 1e-30) + b2_ref[...]
    dinv_b = lax.rsqrt(1.0 + db[:, 16:32])
    hc = jnp.dot(h2, wc_ref[...], preferred_element_type=F32)
    t3_ref[...] = hc * dinv_b
    self_ref[...] = hc * dinv_b * dinv_b
    dinv_ref[...] = dinv_b[:, 0:1]


def _tc4_body(p3_ref, self_ref, bc_ref, out_ref):
    out_ref[...] = p3_ref[0] + p3_ref[1] + self_ref[...] + bc_ref[...]


def kernel(x, edge_index, W1, a_src1, a_dst1, b1, W2, a_src2, a_dst2, b2, Wc, bc):
    src = edge_index[0].astype(I32)
    dst = edge_index[1].astype(I32)
    src_a = src.reshape(NW, _NCH1, _C1)
    dst_a = dst.reshape(NW, _NCH1, _C1)
    src_b = src.reshape(NW, _NCH2, _C2)
    dst_b = dst.reshape(NW, _NCH2, _C2)

    e8 = jnp.eye(8, dtype=F32)
    asrc_m = (a_src1[:, :, None] * e8[:, None, :]).reshape(128, 8)
    adst_m = (a_dst1[:, :, None] * e8[:, None, :]).reshape(128, 8)
    aa = jnp.concatenate([asrc_m, adst_m], axis=1)
    ex8 = jnp.repeat(e8, 16, axis=1)
    a2 = jnp.concatenate([a_src2.T, a_dst2.T], axis=1)
    smat = jnp.zeros((32, 32), F32).at[16, 0:16].set(1.0).at[17, 16:32].set(1.0)
    b1r = b1.reshape(1, 128)
    b2r = b2.reshape(1, 16)
    bcr = bc.reshape(1, 16)

    grid = (N // _B,)

    tsrc1, tdst1 = pl.pallas_call(
        _tc1_body,
        grid=grid,
        in_specs=[
            pl.BlockSpec((_B, 128), lambda i: (i, 0)),
            pl.BlockSpec((128, 128), lambda i: (0, 0)),
            pl.BlockSpec((128, 16), lambda i: (0, 0)),
        ],
        out_specs=[
            pl.BlockSpec((_B, 144), lambda i: (i, 0)),
            pl.BlockSpec((_B, 16), lambda i: (i, 0)),
        ],
        out_shape=[
            jax.ShapeDtypeStruct((N, 144), F32),
            jax.ShapeDtypeStruct((N, 16), F32),
        ],
    )(x, W1, aa)

    p1 = _sc_conv1(src_a, dst_a, tsrc1, tdst1)

    xw2, as2, ad2 = pl.pallas_call(
        _tc2_body,
        grid=grid,
        in_specs=[
            pl.BlockSpec((NC, _B, 144), lambda i: (0, i, 0)),
            pl.BlockSpec((1, 128), lambda i: (0, 0)),
            pl.BlockSpec((128, 16), lambda i: (0, 0)),
            pl.BlockSpec((16, 2), lambda i: (0, 0)),
            pl.BlockSpec((8, 128), lambda i: (0, 0)),
        ],
        out_specs=[
            pl.BlockSpec((_B, 16), lambda i: (i, 0)),
            pl.BlockSpec((_B, 1), lambda i: (i, 0)),
            pl.BlockSpec((_B, 1), lambda i: (i, 0)),
        ],
        out_shape=[
            jax.ShapeDtypeStruct((N, 16), F32),
            jax.ShapeDtypeStruct((N, 1), F32),
            jax.ShapeDtypeStruct((N, 1), F32),
        ],
    )(p1, b1r, W2, a2, ex8)

    p2 = _sc_conv2(src_b, dst_b, xw2, as2.reshape(N), ad2.reshape(N))

    t3, selfterm, dinv = pl.pallas_call(
        _tc3_body,
        grid=grid,
        in_specs=[
            pl.BlockSpec((NC, _B, 32), lambda i: (0, i, 0)),
            pl.BlockSpec((1, 16), lambda i: (0, 0)),
            pl.BlockSpec((16, 16), lambda i: (0, 0)),
            pl.BlockSpec((32, 32), lambda i: (0, 0)),
        ],
        out_specs=[
            pl.BlockSpec((_B, 16), lambda i: (i, 0)),
            pl.BlockSpec((_B, 16), lambda i: (i, 0)),
            pl.BlockSpec((_B, 1), lambda i: (i, 0)),
        ],
        out_shape=[
            jax.ShapeDtypeStruct((N, 16), F32),
            jax.ShapeDtypeStruct((N, 16), F32),
            jax.ShapeDtypeStruct((N, 1), F32),
        ],
    )(p2, b2r, Wc, smat)

    p3 = _sc_iconv(src_b, dst_b, t3, dinv.reshape(N))

    out = pl.pallas_call(
        _tc4_body,
        grid=grid,
        in_specs=[
            pl.BlockSpec((NC, _B, 16), lambda i: (0, i, 0)),
            pl.BlockSpec((_B, 16), lambda i: (i, 0)),
            pl.BlockSpec((1, 16), lambda i: (0, 0)),
        ],
        out_specs=pl.BlockSpec((_B, 16), lambda i: (i, 0)),
        out_shape=jax.ShapeDtypeStruct((N, 16), F32),
    )(p3, selfterm, bcr)
    return out


# speedup vs baseline: 1.3659x; 1.1258x over previous
import functools

import jax
import jax.numpy as jnp
from jax import lax
from jax.experimental import pallas as pl
from jax.experimental.pallas import tpu as pltpu
from jax.experimental.pallas import tpu_sc as plsc

N = 10000
E = 320000
NC = 2
NS = 16
NW = NC * NS
EW = E // NW
ROWS_PER_TILE = N // NS

F32 = jnp.float32
I32 = jnp.int32


_TAKE_DNUMS = lax.GatherDimensionNumbers(
    offset_dims=(), collapsed_slice_dims=(0,), start_index_map=(0,))


def _take16(v, lane):
    idx = jnp.full((16, 1), lane, dtype=I32)
    return lax.gather(v, idx, _TAKE_DNUMS, slice_sizes=(1,),
                      mode=lax.GatherScatterMode.PROMISE_IN_BOUNDS)


def _iota16():
    return lax.iota(I32, 16)


_C1 = 80
_NCH1 = EW // _C1
_I1 = 41


def _sc_conv1_body(srcr, dstr, tsrc, tdst, out,
                   sidxb, didxb, sb0, sb1, sb2, db0, db1, db2, accum,
                   gs0, gs1, gs2, ds0, ds1, ds2, ss0, ss1, ss2,
                   is0, is1, is2, is3, is4, is5, zsem):
    c = lax.axis_index("c")
    s = lax.axis_index("s")
    wid = s * NC + c
    sbufs = (sb0, sb1, sb2)
    dbufs = (db0, db1, db2)
    gsems = (gs0, gs1, gs2)
    dsems = (ds0, ds1, ds2)
    ssems = (ss0, ss1, ss2)
    isems = (is0, is1, is2, is3, is4, is5)

    def i_start(ci, j):
        pltpu.async_copy(srcr.at[wid, ci], sidxb.at[j], isems[j])
        pltpu.async_copy(dstr.at[wid, ci], didxb.at[j], isems[j])

    def i_wait(ci, j):
        pltpu.make_async_copy(srcr.at[wid, ci], sidxb.at[j], isems[j]).wait()
        pltpu.make_async_copy(dstr.at[wid, ci], didxb.at[j], isems[j]).wait()

    def g_start(ci, k, j):
        pltpu.async_copy(tsrc.at[sidxb.at[j]], sbufs[k], gsems[k])
        pltpu.async_copy(tdst.at[didxb.at[j]], dbufs[k], dsems[k])

    def g_wait(ci, k, j):
        pltpu.make_async_copy(tsrc.at[sidxb.at[j]], sbufs[k], gsems[k]).wait()
        pltpu.make_async_copy(tdst.at[didxb.at[j]], dbufs[k], dsems[k]).wait()

    def s_start(ci, k, j):
        pltpu.async_copy(sbufs[k], accum.at[didxb.at[j]], ssems[k], add=True)

    def s_wait(ci, k, j):
        pltpu.make_async_copy(sbufs[k], accum.at[didxb.at[j]], ssems[k]).wait()

    def compute(k):
        sb = sbufs[k]
        db = dbufs[k]

        @plsc.parallel_loop(0, _C1, unroll=4)
        def _(e):
            a_s = sb[e, pl.ds(128, 16)]
            ad = db[e, pl.ds(0, 16)]
            t = a_s + ad
            w = jnp.exp(jnp.maximum(t, 0.2 * t))
            sb[e, pl.ds(128, 16)] = w
            for h in range(8):
                sp = _take16(w, h)
                sb[e, pl.ds(h * 16, 16)] = sb[e, pl.ds(h * 16, 16)] * sp

    i_start(0, 0)
    i_start(1, 1)
    i_start(2, 2)
    z16 = jnp.zeros((16,), F32)
    for e in range(_C1):
        for l in range(9):
            sb2[e, pl.ds(l * 16, 16)] = z16

    @pl.when(s < 15)
    def _():
        for j in range(8):
            pltpu.async_copy(sb2, accum.at[pl.ds(s * 640 + j * 80, 80)], zsem)
        for j in range(8):
            pltpu.make_async_copy(sb2, accum.at[pl.ds(s * 640 + j * 80, 80)], zsem).wait()

    @pl.when(s == 15)
    def _():
        for j in range(5):
            pltpu.async_copy(sb2, accum.at[pl.ds(9600 + j * 80, 80)], zsem)
        for j in range(5):
            pltpu.make_async_copy(sb2, accum.at[pl.ds(9600 + j * 80, 80)], zsem).wait()

    i_wait(0, 0)
    g_start(0, 0, 0)
    plsc.subcore_barrier()

    def do_third(a, i, S, P):
        i_start(a + 3, P[0])

        @pl.when(i > 0)
        def _():
            s_wait(a - 2, 1, P[1])

        i_start(a + 4, P[1])
        i_wait(a + 1, S[1])
        g_start(a + 1, 1, S[1])
        g_wait(a, 0, S[0])
        compute(0)
        s_start(a, 0, S[0])

        @pl.when(i > 0)
        def _():
            s_wait(a - 1, 2, P[2])

        @pl.when(i < _I1 - 1)
        def _():
            i_start(a + 5, P[2])

        i_wait(a + 2, S[2])
        g_start(a + 2, 2, S[2])
        g_wait(a + 1, 1, S[1])
        compute(1)
        s_start(a + 1, 1, S[1])
        s_wait(a, 0, S[0])
        i_wait(a + 3, P[0])
        g_start(a + 3, 0, P[0])
        g_wait(a + 2, 2, S[2])
        compute(2)
        s_start(a + 2, 2, S[2])

    def body(i, carry):
        a = 3 * i
        par = lax.rem(i, 2)

        @pl.when(par == 0)
        def _():
            do_third(a, i, (0, 1, 2), (3, 4, 5))

        @pl.when(par == 1)
        def _():
            do_third(a, i, (3, 4, 5), (0, 1, 2))

        return carry

    lax.fori_loop(0, _I1, body, 0)

    s_wait(121, 1, 1)
    i_wait(124, 4)
    g_start(124, 1, 4)
    g_wait(123, 0, 3)
    compute(0)
    s_start(123, 0, 3)
    g_wait(124, 1, 4)
    compute(1)
    s_start(124, 1, 4)
    s_wait(122, 2, 2)
    s_wait(123, 0, 3)
    s_wait(124, 1, 4)

    plsc.subcore_barrier()

    @pl.when(s < 15)
    def _():
        pltpu.sync_copy(accum.at[pl.ds(s * 640, 640)],
                        out.at[c, pl.ds(s * 640, 640)])

    @pl.when(s == 15)
    def _():
        pltpu.sync_copy(accum.at[pl.ds(9600, 400)],
                        out.at[c, pl.ds(9600, 400)])


def _sc_conv1(srcr, dstr, tsrc, tdst):
    mesh = plsc.VectorSubcoreMesh(core_axis_name="c", subcore_axis_name="s",
                                  num_cores=NC, num_subcores=NS)
    f = functools.partial(
        pl.kernel,
        out_type=jax.ShapeDtypeStruct((NC, N, 144), F32),
        mesh=mesh,
        compiler_params=pltpu.CompilerParams(use_tc_tiling_on_sc=False, needs_layout_passes=False),
        scratch_types=[
            pltpu.VMEM((6, _C1), I32),
            pltpu.VMEM((6, _C1), I32),
            pltpu.VMEM((_C1, 144), F32),
            pltpu.VMEM((_C1, 144), F32),
            pltpu.VMEM((_C1, 144), F32),
            pltpu.VMEM((_C1, 16), F32),
            pltpu.VMEM((_C1, 16), F32),
            pltpu.VMEM((_C1, 16), F32),
            pltpu.VMEM_SHARED((N, 144), F32),
            pltpu.SemaphoreType.DMA,
            pltpu.SemaphoreType.DMA,
            pltpu.SemaphoreType.DMA,
            pltpu.SemaphoreType.DMA,
            pltpu.SemaphoreType.DMA,
            pltpu.SemaphoreType.DMA,
            pltpu.SemaphoreType.DMA,
            pltpu.SemaphoreType.DMA,
            pltpu.SemaphoreType.DMA,
            pltpu.SemaphoreType.DMA,
            pltpu.SemaphoreType.DMA,
            pltpu.SemaphoreType.DMA,
            pltpu.SemaphoreType.DMA,
            pltpu.SemaphoreType.DMA,
            pltpu.SemaphoreType.DMA,
            pltpu.SemaphoreType.DMA,
        ],
    )(_sc_conv1_body)
    return f(srcr, dstr, tsrc, tdst)


_C2 = 400
_NCH2 = EW // _C2
_I2 = 8


def _sc_conv2_body(srcr, dstr, txw, as2h, ad2h, out,
                   sidx, didx, sb0, sb1, sb2, mb0, mb1, mb2, asv, adv, zbuf, accum,
                   gs0, gs1, gs2, ss0, ss1, ss2, zsem):
    c = lax.axis_index("c")
    s = lax.axis_index("s")
    wid = s * NC + c
    sbufs = (sb0, sb1, sb2)
    mbufs = (mb0, mb1, mb2)
    gsems = (gs0, gs1, gs2)
    ssems = (ss0, ss1, ss2)

    pltpu.sync_copy(srcr.at[wid], sidx)
    pltpu.sync_copy(dstr.at[wid], didx)
    pltpu.sync_copy(as2h, asv)
    pltpu.sync_copy(ad2h, adv)

    z16 = jnp.zeros((16,), F32)
    for r in range(40):
        for l in range(2):
            zbuf[r, pl.ds(l * 16, 16)] = z16

    @pl.when(s < 15)
    def _():
        for j in range(16):
            pltpu.async_copy(zbuf, accum.at[pl.ds(s * 640 + j * 40, 40)], zsem)
        for j in range(16):
            pltpu.make_async_copy(zbuf, accum.at[pl.ds(s * 640 + j * 40, 40)], zsem).wait()

    @pl.when(s == 15)
    def _():
        for j in range(10):
            pltpu.async_copy(zbuf, accum.at[pl.ds(9600 + j * 40, 40)], zsem)
        for j in range(10):
            pltpu.make_async_copy(zbuf, accum.at[pl.ds(9600 + j * 40, 40)], zsem).wait()

    def g_start(ci, k):
        pltpu.async_copy(txw.at[sidx.at[ci]], sbufs[k], gsems[k])

    def g_wait(ci, k):
        pltpu.make_async_copy(txw.at[sidx.at[ci]], sbufs[k], gsems[k]).wait()

    def s_start(ci, k):
        pltpu.async_copy(mbufs[k], accum.at[didx.at[ci]], ssems[k], add=True)

    def s_wait(ci, k):
        pltpu.make_async_copy(mbufs[k], accum.at[didx.at[ci]], ssems[k]).wait()

    iota = _iota16()
    oh0 = jnp.where(iota == 0, 1.0, 0.0).astype(F32)
    oh1 = jnp.where(iota == 1, 1.0, 0.0).astype(F32)

    def compute(ci, k):
        sb = sbufs[k]
        mb = mbufs[k]
        ci16 = jnp.full((16,), ci, dtype=I32)

        @plsc.parallel_loop(0, _C2 // 16, unroll=2)
        def _(grp):
            col = iota + grp * 16
            s16 = plsc.load_gather(sidx, [ci16, col])
            d16 = plsc.load_gather(didx, [ci16, col])
            a_s = plsc.load_gather(asv, [s16])
            a_d = plsc.load_gather(adv, [d16])
            t = a_s + a_d
            w = jnp.exp(jnp.maximum(t, 0.2 * t))
            for e in range(16):
                r = grp * 16 + e
                sp = _take16(w, e)
                mb[r, pl.ds(0, 16)] = sb[r, pl.ds(0, 16)] * sp
                mb[r, pl.ds(16, 16)] = sp * oh0 + oh1

    g_start(0, 0)
    plsc.subcore_barrier()

    def body(i, carry):
        a = 3 * i

        @pl.when(i > 0)
        def _():
            s_wait(a - 2, 1)

        g_start(a + 1, 1)
        g_wait(a, 0)
        compute(a, 0)
        s_start(a, 0)

        @pl.when(i > 0)
        def _():
            s_wait(a - 1, 2)

        g_start(a + 2, 2)
        g_wait(a + 1, 1)
        compute(a + 1, 1)
        s_start(a + 1, 1)

        s_wait(a, 0)
        g_start(a + 3, 0)
        g_wait(a + 2, 2)
        compute(a + 2, 2)
        s_start(a + 2, 2)
        return carry

    lax.fori_loop(0, _I2, body, 0)

    last = 3 * _I2
    g_wait(last, 0)
    compute(last, 0)
    s_start(last, 0)
    s_wait(last - 2, 1)
    s_wait(last - 1, 2)
    s_wait(last, 0)

    plsc.subcore_barrier()

    @pl.when(s < 15)
    def _():
        pltpu.sync_copy(accum.at[pl.ds(s * 640, 640)],
                        out.at[c, pl.ds(s * 640, 640)])

    @pl.when(s == 15)
    def _():
        pltpu.sync_copy(accum.at[pl.ds(9600, 400)],
                        out.at[c, pl.ds(9600, 400)])


def _sc_conv2(srcr, dstr, txw, as2, ad2):
    mesh = plsc.VectorSubcoreMesh(core_axis_name="c", subcore_axis_name="s",
                                  num_cores=NC, num_subcores=NS)
    f = functools.partial(
        pl.kernel,
        out_type=jax.ShapeDtypeStruct((NC, N, 32), F32),
        mesh=mesh,
        compiler_params=pltpu.CompilerParams(use_tc_tiling_on_sc=False, needs_layout_passes=False),
        scratch_types=[
            pltpu.VMEM((_NCH2, _C2), I32),
            pltpu.VMEM((_NCH2, _C2), I32),
            pltpu.VMEM((_C2, 16), F32),
            pltpu.VMEM((_C2, 16), F32),
            pltpu.VMEM((_C2, 16), F32),
            pltpu.VMEM((_C2, 32), F32),
            pltpu.VMEM((_C2, 32), F32),
            pltpu.VMEM((_C2, 32), F32),
            pltpu.VMEM((N,), F32),
            pltpu.VMEM((N,), F32),
            pltpu.VMEM((40, 32), F32),
            pltpu.VMEM_SHARED((N, 32), F32),
            pltpu.SemaphoreType.DMA,
            pltpu.SemaphoreType.DMA,
            pltpu.SemaphoreType.DMA,
            pltpu.SemaphoreType.DMA,
            pltpu.SemaphoreType.DMA,
            pltpu.SemaphoreType.DMA,
            pltpu.SemaphoreType.DMA,
        ],
    )(_sc_conv2_body)
    return f(srcr, dstr, txw, as2, ad2)


def _sc_iconv_body(srcr, dstr, t3, dinvh, out,
                   sidx, didx, sb0, sb1, sb2, dinvv, zbuf, accum,
                   gs0, gs1, gs2, ss0, ss1, ss2, zsem):
    c = lax.axis_index("c")
    s = lax.axis_index("s")
    wid = s * NC + c
    sbufs = (sb0, sb1, sb2)
    gsems = (gs0, gs1, gs2)
    ssems = (ss0, ss1, ss2)

    pltpu.sync_copy(srcr.at[wid], sidx)
    pltpu.sync_copy(dstr.at[wid], didx)
    pltpu.sync_copy(dinvh, dinvv)

    z16 = jnp.zeros((16,), F32)
    for r in range(40):
        zbuf[r, pl.ds(0, 16)] = z16

    @pl.when(s < 15)
    def _():
        for j in range(16):
            pltpu.async_copy(zbuf, accum.at[pl.ds(s * 640 + j * 40, 40)], zsem)
        for j in range(16):
            pltpu.make_async_copy(zbuf, accum.at[pl.ds(s * 640 + j * 40, 40)], zsem).wait()

    @pl.when(s == 15)
    def _():
        for j in range(10):
            pltpu.async_copy(zbuf, accum.at[pl.ds(9600 + j * 40, 40)], zsem)
        for j in range(10):
            pltpu.make_async_copy(zbuf, accum.at[pl.ds(9600 + j * 40, 40)], zsem).wait()

    def g_start(ci, k):
        pltpu.async_copy(t3.at[sidx.at[ci]], sbufs[k], gsems[k])

    def g_wait(ci, k):
        pltpu.make_async_copy(t3.at[sidx.at[ci]], sbufs[k], gsems[k]).wait()

    def s_start(ci, k):
        pltpu.async_copy(sbufs[k], accum.at[didx.at[ci]], ssems[k], add=True)

    def s_wait(ci, k):
        pltpu.make_async_copy(sbufs[k], accum.at[didx.at[ci]], ssems[k]).wait()

    iota = _iota16()

    def compute(ci, k):
        sb = sbufs[k]
        ci16 = jnp.full((16,), ci, dtype=I32)

        @plsc.parallel_loop(0, _C2 // 16, unroll=2)
        def _(grp):
            col = iota + grp * 16
            d16 = plsc.load_gather(didx, [ci16, col])
            dd = plsc.load_gather(dinvv, [d16])
            for e in range(16):
                r = grp * 16 + e
                sp = _take16(dd, e)
                sb[r, pl.ds(0, 16)] = sb[r, pl.ds(0, 16)] * sp

    g_start(0, 0)
    plsc.subcore_barrier()

    def body(i, carry):
        a = 3 * i

        @pl.when(i > 0)
        def _():
            s_wait(a - 2, 1)

        g_start(a + 1, 1)
        g_wait(a, 0)
        compute(a, 0)
        s_start(a, 0)

        @pl.when(i > 0)
        def _():
            s_wait(a - 1, 2)

        g_start(a + 2, 2)
        g_wait(a + 1, 1)
        compute(a + 1, 1)
        s_start(a + 1, 1)

        s_wait(a, 0)
        g_start(a + 3, 0)
        g_wait(a + 2, 2)
        compute(a + 2, 2)
        s_start(a + 2, 2)
        return carry

    lax.fori_loop(0, _I2, body, 0)

    last = 3 * _I2
    g_wait(last, 0)
    compute(last, 0)
    s_start(last, 0)
    s_wait(last - 2, 1)
    s_wait(last - 1, 2)
    s_wait(last, 0)

    plsc.subcore_barrier()

    @pl.when(s < 15)
    def _():
        pltpu.sync_copy(accum.at[pl.ds(s * 640, 640)],
                        out.at[c, pl.ds(s * 640, 640)])

    @pl.when(s == 15)
    def _():
        pltpu.sync_copy(accum.at[pl.ds(9600, 400)],
                        out.at[c, pl.ds(9600, 400)])


def _sc_iconv(srcr, dstr, t3, dinv):
    mesh = plsc.VectorSubcoreMesh(core_axis_name="c", subcore_axis_name="s",
                                  num_cores=NC, num_subcores=NS)
    f = functools.partial(
        pl.kernel,
        out_type=jax.ShapeDtypeStruct((NC, N, 16), F32),
        mesh=mesh,
        compiler_params=pltpu.CompilerParams(use_tc_tiling_on_sc=False, needs_layout_passes=False),
        scratch_types=[
            pltpu.VMEM((_NCH2, _C2), I32),
            pltpu.VMEM((_NCH2, _C2), I32),
            pltpu.VMEM((_C2, 16), F32),
            pltpu.VMEM((_C2, 16), F32),
            pltpu.VMEM((_C2, 16), F32),
            pltpu.VMEM((N,), F32),
            pltpu.VMEM((40, 16), F32),
            pltpu.VMEM_SHARED((N, 16), F32),
            pltpu.SemaphoreType.DMA,
            pltpu.SemaphoreType.DMA,
            pltpu.SemaphoreType.DMA,
            pltpu.SemaphoreType.DMA,
            pltpu.SemaphoreType.DMA,
            pltpu.SemaphoreType.DMA,
            pltpu.SemaphoreType.DMA,
        ],
    )(_sc_iconv_body)
    return f(srcr, dstr, t3, dinv)


_B = 1000


def _tc1_body(x_ref, w1_ref, aa_ref, tsrc_ref, tdst_ref):
    xw = jnp.dot(x_ref[...], w1_ref[...], preferred_element_type=F32)
    asad = jnp.dot(xw, aa_ref[...], preferred_element_type=F32)
    tsrc_ref[:, 0:128] = xw
    tsrc_ref[:, 128:136] = asad[:, 0:8]
    tsrc_ref[:, 136:144] = jnp.zeros((_B, 8), F32)
    tdst_ref[:, 0:8] = asad[:, 8:16]
    tdst_ref[:, 8:16] = jnp.zeros((_B, 8), F32)


def _tc2_body(p_ref, b1_ref, w2_ref, a2_ref, ex_ref, xw2_ref, as2_ref, ad2_ref):
    acc = p_ref[0] + p_ref[1]
    num = acc[:, 0:128]
    den = acc[:, 128:136]
    den_b = jnp.dot(den, ex_ref[...], preferred_element_type=F32)
    h1 = num / (den_b + 1e-30) + b1_ref[...]
    h1 = jnp.where(h1 > 0, h1, jnp.exp(jnp.minimum(h1, 0.0)) - 1.0)
    xw2 = jnp.dot(h1, w2_ref[...], preferred_element_type=F32)
    asad2 = jnp.dot(xw2, a2_ref[...], preferred_element_type=F32)
    xw2_ref[...] = xw2
    as2_ref[...] = asad2[:, 0:1]
    ad2_ref[...] = asad2[:, 1:2]


def _tc3_body(p2_ref, b2_ref, wc_ref, s_ref, t3_ref, self_ref, dinv_ref):
    acc = p2_ref[0] + p2_ref[1]
    db = jnp.dot(acc, s_ref[...], preferred_element_type=F32)
    h2 = acc[:, 0:16] / (db[:, 0:16] + 1e-30) + b2_ref[...]
    dinv_b = lax.rsqrt(1.0 + db[:, 16:32])
    hc = jnp.dot(h2, wc_ref[...], preferred_element_type=F32)
    t3_ref[...] = hc * dinv_b
    self_ref[...] = hc * dinv_b * dinv_b
    dinv_ref[...] = dinv_b[:, 0:1]


def _tc4_body(p3_ref, self_ref, bc_ref, out_ref):
    out_ref[...] = p3_ref[0] + p3_ref[1] + self_ref[...] + bc_ref[...]


def kernel(x, edge_index, W1, a_src1, a_dst1, b1, W2, a_src2, a_dst2, b2, Wc, bc):
    src = edge_index[0].astype(I32)
    dst = edge_index[1].astype(I32)
    src_a = src.reshape(NW, _NCH1, _C1)
    dst_a = dst.reshape(NW, _NCH1, _C1)
    src_b = src.reshape(NW, _NCH2, _C2)
    dst_b = dst.reshape(NW, _NCH2, _C2)

    e8 = jnp.eye(8, dtype=F32)
    asrc_m = (a_src1[:, :, None] * e8[:, None, :]).reshape(128, 8)
    adst_m = (a_dst1[:, :, None] * e8[:, None, :]).reshape(128, 8)
    aa = jnp.concatenate([asrc_m, adst_m], axis=1)
    ex8 = jnp.repeat(e8, 16, axis=1)
    a2 = jnp.concatenate([a_src2.T, a_dst2.T], axis=1)
    smat = jnp.zeros((32, 32), F32).at[16, 0:16].set(1.0).at[17, 16:32].set(1.0)
    b1r = b1.reshape(1, 128)
    b2r = b2.reshape(1, 16)
    bcr = bc.reshape(1, 16)

    grid = (N // _B,)

    tsrc1, tdst1 = pl.pallas_call(
        _tc1_body,
        grid=grid,
        in_specs=[
            pl.BlockSpec((_B, 128), lambda i: (i, 0)),
            pl.BlockSpec((128, 128), lambda i: (0, 0)),
            pl.BlockSpec((128, 16), lambda i: (0, 0)),
        ],
        out_specs=[
            pl.BlockSpec((_B, 144), lambda i: (i, 0)),
            pl.BlockSpec((_B, 16), lambda i: (i, 0)),
        ],
        out_shape=[
            jax.ShapeDtypeStruct((N, 144), F32),
            jax.ShapeDtypeStruct((N, 16), F32),
        ],
    )(x, W1, aa)

    p1 = _sc_conv1(src_a, dst_a, tsrc1, tdst1)

    xw2, as2, ad2 = pl.pallas_call(
        _tc2_body,
        grid=grid,
        in_specs=[
            pl.BlockSpec((NC, _B, 144), lambda i: (0, i, 0)),
            pl.BlockSpec((1, 128), lambda i: (0, 0)),
            pl.BlockSpec((128, 16), lambda i: (0, 0)),
            pl.BlockSpec((16, 2), lambda i: (0, 0)),
            pl.BlockSpec((8, 128), lambda i: (0, 0)),
        ],
        out_specs=[
            pl.BlockSpec((_B, 16), lambda i: (i, 0)),
            pl.BlockSpec((_B, 1), lambda i: (i, 0)),
            pl.BlockSpec((_B, 1), lambda i: (i, 0)),
        ],
        out_shape=[
            jax.ShapeDtypeStruct((N, 16), F32),
            jax.ShapeDtypeStruct((N, 1), F32),
            jax.ShapeDtypeStruct((N, 1), F32),
        ],
    )(p1, b1r, W2, a2, ex8)

    p2 = _sc_conv2(src_b, dst_b, xw2, as2.reshape(N), ad2.reshape(N))

    t3, selfterm, dinv = pl.pallas_call(
        _tc3_body,
        grid=grid,
        in_specs=[
            pl.BlockSpec((NC, _B, 32), lambda i: (0, i, 0)),
            pl.BlockSpec((1, 16), lambda i: (0, 0)),
            pl.BlockSpec((16, 16), lambda i: (0, 0)),
            pl.BlockSpec((32, 32), lambda i: (0, 0)),
        ],
        out_specs=[
            pl.BlockSpec((_B, 16), lambda i: (i, 0)),
            pl.BlockSpec((_B, 16), lambda i: (i, 0)),
            pl.BlockSpec((_B, 1), lambda i: (i, 0)),
        ],
        out_shape=[
            jax.ShapeDtypeStruct((N, 16), F32),
            jax.ShapeDtypeStruct((N, 16), F32),
            jax.ShapeDtypeStruct((N, 1), F32),
        ],
    )(p2, b2r, Wc, smat)

    p3 = _sc_iconv(src_b, dst_b, t3, dinv.reshape(N))

    out = pl.pallas_call(
        _tc4_body,
        grid=grid,
        in_specs=[
            pl.BlockSpec((NC, _B, 16), lambda i: (0, i, 0)),
            pl.BlockSpec((_B, 16), lambda i: (i, 0)),
            pl.BlockSpec((1, 16), lambda i: (0, 0)),
        ],
        out_specs=pl.BlockSpec((_B, 16), lambda i: (i, 0)),
        out_shape=jax.ShapeDtypeStruct((N, 16), F32),
    )(p3, selfterm, bcr)
    return out
